# Initial kernel scaffold; baseline (speedup 1.0000x reference)
import jax
import jax.numpy as jnp
from jax.experimental import pallas as pl


def kernel(x, edge_attr, params, edge_index, batch):
    raise NotImplementedError("write your pallas kernel here")


# speedup vs baseline: 2.2703x; 2.2703x over previous
import functools

import jax
import jax.numpy as jnp
import numpy as np
from jax.experimental import pallas as pl
from jax.experimental.pallas import tpu as pltpu

N = 10000
E = 160000
NG = 128
HIDDEN = 64
TOWERS = 4
F_OUT_T = HIDDEN // TOWERS
AVG_LOG = float(np.log(17.0))


def _layer_norm(x, g, b):
    mu = jnp.mean(x, axis=-1, keepdims=True)
    var = jnp.mean((x - mu) ** 2, axis=-1, keepdims=True)
    return (x - mu) / jnp.sqrt(var + 1e-5) * g + b


def _head_kernel(g_ref, l1w, l1b, ln1g, ln1b, l2w, l2b, ln2g, ln2b,
                 o1w, o1b, o2w, o2b, o3w, o3b, out_ref):
    g = g_ref[...]
    g = jax.nn.relu(_layer_norm(g @ l1w[...] + l1b[...], ln1g[...], ln1b[...]))
    g = jax.nn.relu(_layer_norm(g @ l2w[...] + l2b[...], ln2g[...], ln2b[...]))
    g = jax.nn.relu(g @ o1w[...] + o1b[...])
    g = jax.nn.relu(g @ o2w[...] + o2b[...])
    g = g @ o3w[...] + o3b[...]
    out_ref[...] = jnp.abs(g)


def _head(g, m, o):
    args = (g, m["l1"]["W"], m["l1"]["b"], m["ln1_g"], m["ln1_b"],
            m["l2"]["W"], m["l2"]["b"], m["ln2_g"], m["ln2_b"],
            o["o1"]["W"], o["o1"]["b"], o["o2"]["W"], o["o2"]["b"],
            o["o3"]["W"], o["o3"]["b"])
    return pl.pallas_call(
        _head_kernel,
        out_shape=jax.ShapeDtypeStruct((NG, 3), jnp.float32),
    )(*args)


def _conv_layer(c, x, edge_attr, src, dst, cnt, has, amp, att):
    fin = x.shape[1]
    K = TOWERS * fin

    Wpre = jnp.concatenate([c["pre"][t]["W"] for t in range(TOWERS)], axis=1)
    bpre = jnp.concatenate([c["pre"][t]["b"] for t in range(TOWERS)], axis=0)
    Wd, Ws, We2 = Wpre[:fin], Wpre[fin:2 * fin], Wpre[2 * fin:]
    M3 = c["edge"]["W"] @ We2
    c0 = c["edge"]["b"] @ We2

    A = x @ Wd + bpre + c0
    B = x @ Ws

    m = B[src] + edge_attr @ M3
    S = jax.ops.segment_sum(m, dst, num_segments=N)
    Q = jax.ops.segment_sum(m * m, dst, num_segments=N)
    mn = jax.ops.segment_min(m, dst, num_segments=N)
    mx = jax.ops.segment_max(m, dst, num_segments=N)

    cnt_ = cnt[:, None]
    mean = jnp.where(has, A + S / cnt_, 0.0)
    var = jnp.maximum(Q / cnt_ - (S / cnt_) ** 2, 0.0)
    std = jnp.sqrt(jnp.where(has, var, 0.0) + 1e-5)
    mn = jnp.where(has, A + mn, 0.0)
    mx = jnp.where(has, A + mx, 0.0)

    aggs = []
    for t in range(TOWERS):
        sl = slice(t * fin, (t + 1) * fin)
        aggs.append(jnp.concatenate(
            [mean[:, sl], mn[:, sl], mx[:, sl], std[:, sl]], axis=1))

    Wlin = c["lin"]["W"]
    acc = jnp.zeros((x.shape[0], HIDDEN), jnp.float32)
    bias = c["lin"]["b"]
    Px = jnp.zeros((fin, HIDDEN), jnp.float32)
    for t in range(TOWERS):
        Wl_t = Wlin[t * F_OUT_T:(t + 1) * F_OUT_T]
        Wp = c["post"][t]["W"]
        Px = Px + Wp[:fin] @ Wl_t
        W1 = Wp[fin:5 * fin] @ Wl_t
        W2 = Wp[5 * fin:9 * fin] @ Wl_t
        W3 = Wp[9 * fin:] @ Wl_t
        a = aggs[t]
        acc = acc + a @ W1 + amp * (a @ W2) + att * (a @ W3)
        bias = bias + c["post"][t]["b"] @ Wl_t
    y = x @ Px + acc + bias
    return jax.nn.relu(_layer_norm(y, c["ln_g"], c["ln_b"]))


def kernel(x, edge_attr, params, edge_index, batch):
    src = edge_index[0]
    dst = edge_index[1]
    count = jax.ops.segment_sum(jnp.ones((E,), jnp.float32), dst, num_segments=N)
    cnt = jnp.clip(count, 1.0)
    logdeg = jnp.log(jnp.clip(count, 1.0) + 1.0)
    amp = (logdeg / AVG_LOG)[:, None]
    att = (AVG_LOG / logdeg)[:, None]
    has = (count > 0)[:, None]

    for c in params["convs"]:
        x = _conv_layer(c, x, edge_attr, src, dst, cnt, has, amp, att)

    g = jax.ops.segment_sum(x, batch, num_segments=NG)
    return _head(g, params["mlp"], params["out"])


# speedup vs baseline: 2.3703x; 1.0440x over previous
import functools

import jax
import jax.numpy as jnp
import numpy as np
from jax.experimental import pallas as pl
from jax.experimental.pallas import tpu as pltpu

N = 10000
E = 160000
NG = 128
HIDDEN = 64
TOWERS = 4
F_OUT_T = HIDDEN // TOWERS
AVG_LOG = float(np.log(17.0))
BLK = 2000
GRID = N // BLK


def _layer_norm(x, g, b):
    mu = jnp.mean(x, axis=-1, keepdims=True)
    var = jnp.mean((x - mu) ** 2, axis=-1, keepdims=True)
    return (x - mu) / jnp.sqrt(var + 1e-5) * g + b


def _proj_kernel(xb, Wd, Ws, We2, be, bpre, We, A_out, B_out, M3_out):
    c0 = be[...] @ We2[...]
    A_out[...] = xb[...] @ Wd[...] + bpre[...] + c0
    B_out[...] = xb[...] @ Ws[...]
    M3_out[...] = We[...] @ We2[...]


def _proj(x, Wd, Ws, We2, be, bpre, We):
    fin = x.shape[1]
    K = Wd.shape[1]
    return pl.pallas_call(
        _proj_kernel,
        grid=(GRID,),
        in_specs=[
            pl.BlockSpec((BLK, fin), lambda i: (i, 0)),
            pl.BlockSpec((fin, K), lambda i: (0, 0)),
            pl.BlockSpec((fin, K), lambda i: (0, 0)),
            pl.BlockSpec((fin, K), lambda i: (0, 0)),
            pl.BlockSpec((fin,), lambda i: (0,)),
            pl.BlockSpec((K,), lambda i: (0,)),
            pl.BlockSpec((3, fin), lambda i: (0, 0)),
        ],
        out_specs=[
            pl.BlockSpec((BLK, K), lambda i: (i, 0)),
            pl.BlockSpec((BLK, K), lambda i: (i, 0)),
            pl.BlockSpec((3, K), lambda i: (0, 0)),
        ],
        out_shape=[
            jax.ShapeDtypeStruct((N, K), jnp.float32),
            jax.ShapeDtypeStruct((N, K), jnp.float32),
            jax.ShapeDtypeStruct((3, K), jnp.float32),
        ],
    )(x, Wd, Ws, We2, be, bpre, We)


def _post_kernel(fin, xb, Ab, Sb, Qb, MNb, MXb, count_ref, Wpost, bpost,
                 Wlin, blin, ln_g, ln_b, out_ref):
    K = 4 * fin
    cntf = count_ref[...]
    has = cntf > 0.0
    cnt = jnp.maximum(cntf, 1.0)
    logdeg = jnp.log(cnt + 1.0)
    amp = logdeg / AVG_LOG
    att = AVG_LOG / logdeg

    A = Ab[...]
    S = Sb[...]
    Q = Qb[...]
    mean = jnp.where(has, A + S / cnt, 0.0)
    var = jnp.maximum(Q / cnt - (S / cnt) ** 2, 0.0)
    std = jnp.sqrt(jnp.where(has, var, 0.0) + 1e-5)
    mn = jnp.where(has, A + MNb[...], 0.0)
    mx = jnp.where(has, A + MXb[...], 0.0)

    y = jnp.zeros((BLK, HIDDEN), jnp.float32)
    bias = blin[...]
    stats = (mean, mn, mx, std)
    for t in range(TOWERS):
        Wl_t = Wlin[t * F_OUT_T:(t + 1) * F_OUT_T, :]
        Wp = Wpost[t]
        Px_t = Wp[0:fin, :] @ Wl_t
        y += xb[...] @ Px_t
        bias += bpost[t] @ Wl_t
        for g in range(3):
            scale = (1.0, amp, att)[g]
            acc = jnp.zeros((BLK, HIDDEN), jnp.float32)
            for s in range(4):
                lo = fin + g * K + s * fin
                Wrows = Wp[lo:lo + fin, :] @ Wl_t
                acc += stats[s][:, t * fin:(t + 1) * fin] @ Wrows
            y = y + acc * scale
    y = y + bias
    out_ref[...] = jax.nn.relu(_layer_norm(y, ln_g[...], ln_b[...]))


def _post(fin, x, A, S, Q, MN, MX, count, Wpost, bpost, Wlin, blin, ln_g, ln_b):
    K = 4 * fin
    return pl.pallas_call(
        functools.partial(_post_kernel, fin),
        grid=(GRID,),
        in_specs=[
            pl.BlockSpec((BLK, fin), lambda i: (i, 0)),
            pl.BlockSpec((BLK, K), lambda i: (i, 0)),
            pl.BlockSpec((BLK, K), lambda i: (i, 0)),
            pl.BlockSpec((BLK, K), lambda i: (i, 0)),
            pl.BlockSpec((BLK, K), lambda i: (i, 0)),
            pl.BlockSpec((BLK, K), lambda i: (i, 0)),
            pl.BlockSpec((BLK, 1), lambda i: (i, 0)),
            pl.BlockSpec((TOWERS, 13 * fin, F_OUT_T), lambda i: (0, 0, 0)),
            pl.BlockSpec((TOWERS, F_OUT_T), lambda i: (0, 0)),
            pl.BlockSpec((HIDDEN, HIDDEN), lambda i: (0, 0)),
            pl.BlockSpec((HIDDEN,), lambda i: (0,)),
            pl.BlockSpec((HIDDEN,), lambda i: (0,)),
            pl.BlockSpec((HIDDEN,), lambda i: (0,)),
        ],
        out_specs=pl.BlockSpec((BLK, HIDDEN), lambda i: (i, 0)),
        out_shape=jax.ShapeDtypeStruct((N, HIDDEN), jnp.float32),
    )(x, A, S, Q, MN, MX, count[:, None], Wpost, bpost,
      Wlin, blin, ln_g, ln_b)


def _post_count_fix(count):
    return count


def _pool_kernel(xb, batch_ref, g_out):
    i = pl.program_id(0)

    @pl.when(i == 0)
    def _():
        g_out[...] = jnp.zeros_like(g_out)

    b = batch_ref[...].reshape(1, BLK)
    onehot = (jax.lax.broadcasted_iota(jnp.int32, (NG, BLK), 0)
              == b).astype(jnp.float32)
    g_out[...] += onehot @ xb[...]


def _pool(x, batch):
    return pl.pallas_call(
        _pool_kernel,
        grid=(GRID,),
        in_specs=[
            pl.BlockSpec((BLK, HIDDEN), lambda i: (i, 0)),
            pl.BlockSpec((1, 1, BLK), lambda i: (i, 0, 0)),
        ],
        out_specs=pl.BlockSpec((NG, HIDDEN), lambda i: (0, 0)),
        out_shape=jax.ShapeDtypeStruct((NG, HIDDEN), jnp.float32),
    )(x, batch.reshape(GRID, 1, BLK))


def _head_kernel(g_ref, l1w, l1b, ln1g, ln1b, l2w, l2b, ln2g, ln2b,
                 o1w, o1b, o2w, o2b, o3w, o3b, out_ref):
    g = g_ref[...]
    g = jax.nn.relu(_layer_norm(g @ l1w[...] + l1b[...], ln1g[...], ln1b[...]))
    g = jax.nn.relu(_layer_norm(g @ l2w[...] + l2b[...], ln2g[...], ln2b[...]))
    g = jax.nn.relu(g @ o1w[...] + o1b[...])
    g = jax.nn.relu(g @ o2w[...] + o2b[...])
    g = g @ o3w[...] + o3b[...]
    out_ref[...] = jnp.abs(g)


def _head(g, m, o):
    args = (g, m["l1"]["W"], m["l1"]["b"], m["ln1_g"], m["ln1_b"],
            m["l2"]["W"], m["l2"]["b"], m["ln2_g"], m["ln2_b"],
            o["o1"]["W"], o["o1"]["b"], o["o2"]["W"], o["o2"]["b"],
            o["o3"]["W"], o["o3"]["b"])
    return pl.pallas_call(
        _head_kernel,
        out_shape=jax.ShapeDtypeStruct((NG, 3), jnp.float32),
    )(*args)


def _conv_layer(c, x, edge_attr, src, dst, count):
    fin = x.shape[1]
    K = TOWERS * fin

    Wpre = jnp.concatenate([c["pre"][t]["W"] for t in range(TOWERS)], axis=1)
    bpre = jnp.concatenate([c["pre"][t]["b"] for t in range(TOWERS)], axis=0)
    Wd, Ws, We2 = Wpre[:fin], Wpre[fin:2 * fin], Wpre[2 * fin:]

    A, B, M3 = _proj(x, Wd, Ws, We2, c["edge"]["b"], bpre, c["edge"]["W"])

    m = B[src] + edge_attr @ M3
    S = jax.ops.segment_sum(m, dst, num_segments=N)
    Q = jax.ops.segment_sum(m * m, dst, num_segments=N)
    MN = jax.ops.segment_min(m, dst, num_segments=N)
    MX = jax.ops.segment_max(m, dst, num_segments=N)
    MN = jnp.where(count[:, None] > 0, MN, 0.0)
    MX = jnp.where(count[:, None] > 0, MX, 0.0)

    Wpost = jnp.stack([c["post"][t]["W"] for t in range(TOWERS)])
    bpost = jnp.stack([c["post"][t]["b"] for t in range(TOWERS)])
    return _post(fin, x, A, S, Q, MN, MX, count, Wpost, bpost,
                 c["lin"]["W"], c["lin"]["b"], c["ln_g"], c["ln_b"])


def kernel(x, edge_attr, params, edge_index, batch):
    src = edge_index[0].astype(jnp.int32)
    dst = edge_index[1].astype(jnp.int32)
    count = jax.ops.segment_sum(jnp.ones((E,), jnp.float32), dst, num_segments=N)

    for c in params["convs"]:
        x = _conv_layer(c, x, edge_attr, src, dst, count)

    g = _pool(x, batch.astype(jnp.int32))
    return _head(g, params["mlp"], params["out"])


# speedup vs baseline: 4.6622x; 1.9669x over previous
import functools

import jax
import jax.numpy as jnp
import numpy as np
from jax import lax
from jax.experimental import pallas as pl
from jax.experimental.pallas import tpu as pltpu
from jax.experimental.pallas import tpu_sc as plsc

N = 10000
E = 160000
NG = 128
HIDDEN = 64
TOWERS = 4
F_OUT_T = HIDDEN // TOWERS
AVG_LOG = float(np.log(17.0))
BLK = 2000
GRID = N // BLK

NTILES = 32
BSHIFT = 5
BW = 1 << BSHIFT
NB = 320
N_PAD = NB * BW
NBPT = NB // NTILES
CH = 5024
E_IN_PAD = NTILES * CH
C3 = 256
E_PAD = E + 264 + 128
DUMP = E + 264

def _wid():
    return lax.axis_index("s") * 2 + lax.axis_index("c")


@functools.cache
def _sc_hist_k():
    return functools.partial(
        pl.kernel,
        out_type=jax.ShapeDtypeStruct((NTILES, NB), jnp.int32),
        mesh=plsc.VectorSubcoreMesh(core_axis_name="c", subcore_axis_name="s"),
        scratch_types=[
            pltpu.VMEM((CH + 16,), jnp.int32),
            pltpu.VMEM((NB,), jnp.int32),
            pltpu.SMEM((NB,), jnp.int32),
        ],
    )(_sc_hist_body)


def _sc_hist_body(dst_hbm, hist_out, dstv, histv, histm):
    w = _wid()
    base = w * CH
    cnt = jnp.minimum(CH, E - base)
    pltpu.sync_copy(dst_hbm.at[pl.ds(base, CH)], dstv.at[pl.ds(0, CH)])

    def z(b, c):
        histm[b] = 0
        return c

    lax.fori_loop(0, NB, z, 0)

    def body(i, carry):
        b = lax.shift_right_logical(dstv[pl.ds(i, 16)][0], BSHIFT)
        histm[b] = histm[b] + 1
        return carry

    lax.fori_loop(0, cnt, body, 0)

    lane = lax.iota(jnp.int32, 16)
    zero16 = jnp.zeros((16,), jnp.int32)

    def red(bg, c):
        tot = zero16
        for l in range(16):
            tot = jnp.where(lane == l, histm[bg * 16 + l], tot)
        histv[pl.ds(bg * 16, 16)] = tot
        return c

    lax.fori_loop(0, NB // 16, red, 0)
    pltpu.sync_copy(histv, hist_out.at[w])


@functools.cache
def _sc_permute_k():
    return functools.partial(
        pl.kernel,
        out_type=[
            jax.ShapeDtypeStruct((E_PAD,), jnp.int32),
            jax.ShapeDtypeStruct((E_PAD,), jnp.int32),
            jax.ShapeDtypeStruct((E_PAD,), jnp.float32),
            jax.ShapeDtypeStruct((E_PAD,), jnp.float32),
            jax.ShapeDtypeStruct((E_PAD,), jnp.float32),
        ],
        mesh=plsc.VectorSubcoreMesh(core_axis_name="c", subcore_axis_name="s"),
        scratch_types=[
            pltpu.VMEM((NTILES, NB), jnp.int32),
            pltpu.SMEM((NB,), jnp.int32),
            pltpu.VMEM((5040,), jnp.int32),
            pltpu.VMEM((5040,), jnp.int32),
            pltpu.VMEM((5040,), jnp.float32),
            pltpu.VMEM((5040,), jnp.float32),
            pltpu.VMEM((5040,), jnp.float32),
            pltpu.VMEM((45, 112), jnp.int32),
            pltpu.VMEM((256,), jnp.float32),
            pltpu.VMEM((256,), jnp.int32),
            pltpu.SemaphoreType.DMA,
        ],
    )(_sc_permute_body)


def _sc_permute_body(src_hbm, dst_hbm, a0_hbm, a1_hbm, a2_hbm, hist_hbm,
                srcs_out, dsts_out, a0s_out, a1s_out, a2s_out,
                histv, offsm, sv, dv, a0v, a1v, a2v, posv, zbuf, zbi, sem):
    w = _wid()
    base = w * CH
    cnt = jnp.minimum(CH, E - base)
    pltpu.sync_copy(hist_hbm, histv)
    pltpu.sync_copy(src_hbm.at[pl.ds(base, CH)], sv.at[pl.ds(0, CH)])
    pltpu.sync_copy(dst_hbm.at[pl.ds(base, CH)], dv.at[pl.ds(0, CH)])
    pltpu.sync_copy(a0_hbm.at[pl.ds(base, CH)], a0v.at[pl.ds(0, CH)])
    pltpu.sync_copy(a1_hbm.at[pl.ds(base, CH)], a1v.at[pl.ds(0, CH)])
    pltpu.sync_copy(a2_hbm.at[pl.ds(base, CH)], a2v.at[pl.ds(0, CH)])

    lane = lax.iota(jnp.int32, 16)
    zero16 = jnp.zeros((16,), jnp.int32)

    def ob(bg, run):
        tot16 = zero16
        mine16 = zero16

        def iw(w2, c):
            t16, m16 = c
            h16 = histv[w2, pl.ds(bg * 16, 16)]
            return (t16 + h16, m16 + jnp.where(w2 < w, h16, zero16))

        tot16, mine16 = lax.fori_loop(0, NTILES, iw, (tot16, mine16))
        for l in range(16):
            offsm[bg * 16 + l] = run + mine16[l]
            run = run + tot16[l]
        return run

    lax.fori_loop(0, NB // 16, ob, 0)

    dumpv = jnp.full((16,), DUMP, jnp.int32)

    def dump(g, carry):
        cc = g // 7
        gi = g - cc * 7
        posv[cc, pl.ds(gi * 16, 16)] = dumpv
        return carry

    lax.fori_loop(0, 315, dump, 0)

    def grp(g, carry):
        cc = g // 7
        gi = g - cc * 7
        b16 = lax.shift_right_logical(dv[pl.ds(g * 16, 16)], BSHIFT)
        pos16 = zero16
        for l in range(16):
            b = b16[l]
            p = offsm[b]
            offsm[b] = p + 1
            pos16 = jnp.where(lane == l, p, pos16)
        posv[cc, pl.ds(gi * 16, 16)] = pos16
        return carry

    lax.fori_loop(0, lax.shift_right_arithmetic(cnt, 4), grp, 0)

    for c in range(45):
        sl = pl.ds(c * 112, 112)
        idx = posv.at[c]
        d1 = pltpu.async_copy(sv.at[sl], srcs_out.at[idx], sem)
        d2 = pltpu.async_copy(dv.at[sl], dsts_out.at[idx], sem)
        d3 = pltpu.async_copy(a0v.at[sl], a0s_out.at[idx], sem)
        d4 = pltpu.async_copy(a1v.at[sl], a1s_out.at[idx], sem)
        d5 = pltpu.async_copy(a2v.at[sl], a2s_out.at[idx], sem)
        d1.wait(); d2.wait(); d3.wait(); d4.wait(); d5.wait()

    @pl.when(w == 0)
    def _():
        for i in range(16):
            zbuf[pl.ds(i * 16, 16)] = jnp.zeros((16,), jnp.float32)
            zbi[pl.ds(i * 16, 16)] = jnp.zeros((16,), jnp.int32)
        pltpu.sync_copy(zbi, srcs_out.at[pl.ds(E, 256)])
        pltpu.sync_copy(zbi, dsts_out.at[pl.ds(E, 256)])
        pltpu.sync_copy(zbuf, a0s_out.at[pl.ds(E, 256)])
        pltpu.sync_copy(zbuf, a1s_out.at[pl.ds(E, 256)])
        pltpu.sync_copy(zbuf, a2s_out.at[pl.ds(E, 256)])


@functools.cache
def _make_sc_accum(K):
    NJ = K // 16

    @functools.partial(
        pl.kernel,
        out_type=[
            jax.ShapeDtypeStruct((N_PAD, K), jnp.float32),
            jax.ShapeDtypeStruct((N_PAD, K), jnp.float32),
            jax.ShapeDtypeStruct((N_PAD, K), jnp.float32),
            jax.ShapeDtypeStruct((N_PAD, K), jnp.float32),
            jax.ShapeDtypeStruct((N_PAD, 16), jnp.float32),
        ],
        mesh=plsc.VectorSubcoreMesh(core_axis_name="c", subcore_axis_name="s"),
        scratch_types=[
            pltpu.VMEM((NTILES, NB), jnp.int32),
            pltpu.SMEM((NB + 1,), jnp.int32),
            pltpu.VMEM((288,), jnp.int32),
            pltpu.VMEM((288,), jnp.int32),
            pltpu.VMEM((288,), jnp.float32),
            pltpu.VMEM((288,), jnp.float32),
            pltpu.VMEM((288,), jnp.float32),
            pltpu.VMEM((272, K), jnp.float32),
            pltpu.VMEM((3, K), jnp.float32),
            pltpu.VMEM((BW, K), jnp.float32),
            pltpu.VMEM((BW, K), jnp.float32),
            pltpu.VMEM((BW, K), jnp.float32),
            pltpu.VMEM((BW, K), jnp.float32),
            pltpu.VMEM((BW, 16), jnp.float32),
            pltpu.SemaphoreType.DMA,
        ],
    )
    def _sc_accum(srcs_hbm, dsts_hbm, a0_hbm, a1_hbm, a2_hbm, hist_hbm,
                  B_hbm, m3_hbm, S_out, Q_out, MN_out, MX_out, cnt_out,
                  histv, prefm, sv, dv, a0v, a1v, a2v, rowsv, m3v,
                  accS, accQ, accMN, accMX, accC, sem):
        w = _wid()
        pltpu.sync_copy(hist_hbm, histv)
        pltpu.sync_copy(m3_hbm, m3v)

        zero16i = jnp.zeros((16,), jnp.int32)

        def pb(bg, run):
            def iw(w2, t16):
                return t16 + histv[w2, pl.ds(bg * 16, 16)]

            tot16 = lax.fori_loop(0, NTILES, iw, zero16i)
            for l in range(16):
                prefm[bg * 16 + l] = run
                run = run + tot16[l]
            return run

        run = lax.fori_loop(0, NB // 16, pb, 0)
        prefm[NB] = run

        zero16 = jnp.zeros((16,), jnp.float32)
        big16 = jnp.full((16,), 1e30, jnp.float32)
        ones16 = jnp.ones((16,), jnp.float32)

        def tbody(t, tcarry):
            b = w + NTILES * t
            start = prefm[b]
            end = prefm[b + 1]

            def zr(r, carry):
                for j in range(NJ):
                    accS[r, pl.ds(j * 16, 16)] = zero16
                    accQ[r, pl.ds(j * 16, 16)] = zero16
                    accMN[r, pl.ds(j * 16, 16)] = big16
                    accMX[r, pl.ds(j * 16, 16)] = -big16
                accC[r, pl.ds(0, 16)] = zero16
                return carry

            lax.fori_loop(0, BW, zr, 0)

            def chunk_body(k, kcarry):
                cbase = start + k * C3
                a8 = pl.multiple_of(lax.bitwise_and(cbase, ~15), 16)
                off = cbase - a8
                valid = jnp.minimum(C3, end - cbase)
                pltpu.sync_copy(srcs_hbm.at[pl.ds(a8, 288)], sv)
                pltpu.sync_copy(dsts_hbm.at[pl.ds(a8, 288)], dv)
                pltpu.sync_copy(a0_hbm.at[pl.ds(a8, 288)], a0v)
                pltpu.sync_copy(a1_hbm.at[pl.ds(a8, 288)], a1v)
                pltpu.sync_copy(a2_hbm.at[pl.ds(a8, 288)], a2v)
                g1 = pltpu.async_copy(B_hbm.at[sv.at[pl.ds(0, 128)]],
                                      rowsv.at[pl.ds(0, 128)], sem)
                g2 = pltpu.async_copy(B_hbm.at[sv.at[pl.ds(128, 128)]],
                                      rowsv.at[pl.ds(128, 128)], sem)
                g3 = pltpu.async_copy(B_hbm.at[sv.at[pl.ds(256, 16)]],
                                      rowsv.at[pl.ds(256, 16)], sem)
                g1.wait()
                g2.wait()
                g3.wait()

                def edge(i, carry):
                    ii = off + i
                    dl = lax.bitwise_and(dv[pl.ds(ii, 16)][0], BW - 1)
                    a0 = a0v[pl.ds(ii, 16)][0]
                    a1 = a1v[pl.ds(ii, 16)][0]
                    a2 = a2v[pl.ds(ii, 16)][0]
                    for j in range(NJ):
                        sl = pl.ds(j * 16, 16)
                        m = (rowsv[ii, sl] + a0 * m3v[0, sl]
                             + a1 * m3v[1, sl] + a2 * m3v[2, sl])
                        plsc.addupdate(accS.at[dl, sl], m)
                        plsc.addupdate(accQ.at[dl, sl], m * m)
                        accMN[dl, sl] = jnp.minimum(accMN[dl, sl], m)
                        accMX[dl, sl] = jnp.maximum(accMX[dl, sl], m)
                    plsc.addupdate(accC.at[dl, pl.ds(0, 16)], ones16)
                    return carry

                lax.fori_loop(0, valid, edge, 0)
                return kcarry

            nch = lax.shift_right_arithmetic(end - start + C3 - 1, 8)
            lax.fori_loop(0, nch, chunk_body, 0)

            rsl = pl.ds(b * BW, BW)
            pltpu.sync_copy(accS, S_out.at[rsl])
            pltpu.sync_copy(accQ, Q_out.at[rsl])
            pltpu.sync_copy(accMN, MN_out.at[rsl])
            pltpu.sync_copy(accMX, MX_out.at[rsl])
            pltpu.sync_copy(accC, cnt_out.at[rsl])
            return tcarry

        lax.fori_loop(0, NBPT, tbody, 0)

    return _sc_accum


def _layer_norm(x, g, b):
    mu = jnp.mean(x, axis=-1, keepdims=True)
    var = jnp.mean((x - mu) ** 2, axis=-1, keepdims=True)
    return (x - mu) / jnp.sqrt(var + 1e-5) * g + b


def _proj_kernel(xb, Wd, Ws, We2, be, bpre, We, A_out, B_out, M3_out):
    c0 = be[...] @ We2[...]
    A_out[...] = xb[...] @ Wd[...] + bpre[...] + c0
    B_out[...] = xb[...] @ Ws[...]
    M3_out[...] = We[...] @ We2[...]


def _proj(x, Wd, Ws, We2, be, bpre, We):
    fin = x.shape[1]
    Kp = Wd.shape[1]
    return pl.pallas_call(
        _proj_kernel,
        grid=(GRID,),
        in_specs=[
            pl.BlockSpec((BLK, fin), lambda i: (i, 0)),
            pl.BlockSpec((fin, Kp), lambda i: (0, 0)),
            pl.BlockSpec((fin, Kp), lambda i: (0, 0)),
            pl.BlockSpec((fin, Kp), lambda i: (0, 0)),
            pl.BlockSpec((fin,), lambda i: (0,)),
            pl.BlockSpec((Kp,), lambda i: (0,)),
            pl.BlockSpec((3, fin), lambda i: (0, 0)),
        ],
        out_specs=[
            pl.BlockSpec((BLK, Kp), lambda i: (i, 0)),
            pl.BlockSpec((BLK, Kp), lambda i: (i, 0)),
            pl.BlockSpec((3, Kp), lambda i: (0, 0)),
        ],
        out_shape=[
            jax.ShapeDtypeStruct((N, Kp), jnp.float32),
            jax.ShapeDtypeStruct((N, Kp), jnp.float32),
            jax.ShapeDtypeStruct((3, Kp), jnp.float32),
        ],
    )(x, Wd, Ws, We2, be, bpre, We)


def _post_kernel(fin, xb, Ab, Sb, Qb, MNb, MXb, count_ref, Wpost, bpost,
                 Wlin, blin, ln_g, ln_b, out_ref):
    K = 4 * fin
    cntf = count_ref[...][:, 0:1]
    has = cntf > 0.0
    cnt = jnp.maximum(cntf, 1.0)
    logdeg = jnp.log(cnt + 1.0)
    amp = logdeg / AVG_LOG
    att = AVG_LOG / logdeg

    A = Ab[...]
    S = Sb[...]
    Q = Qb[...]
    mean = jnp.where(has, A + S / cnt, 0.0)
    var = jnp.maximum(Q / cnt - (S / cnt) ** 2, 0.0)
    std = jnp.sqrt(jnp.where(has, var, 0.0) + 1e-5)
    mn = jnp.where(has, A + MNb[...], 0.0)
    mx = jnp.where(has, A + MXb[...], 0.0)

    y = jnp.zeros((BLK, HIDDEN), jnp.float32)
    bias = blin[...]
    stats = (mean, mn, mx, std)
    for t in range(TOWERS):
        Wl_t = Wlin[t * F_OUT_T:(t + 1) * F_OUT_T, :]
        Wp = Wpost[t]
        Px_t = Wp[0:fin, :] @ Wl_t
        y += xb[...] @ Px_t
        bias += bpost[t] @ Wl_t
        for g in range(3):
            scale = (1.0, amp, att)[g]
            acc = jnp.zeros((BLK, HIDDEN), jnp.float32)
            for s in range(4):
                lo = fin + g * K + s * fin
                Wrows = Wp[lo:lo + fin, :] @ Wl_t
                acc += stats[s][:, t * fin:(t + 1) * fin] @ Wrows
            y = y + acc * scale
    y = y + bias
    out_ref[...] = jax.nn.relu(_layer_norm(y, ln_g[...], ln_b[...]))


def _post(fin, Kp, x, A, S, Q, MN, MX, count, Wpost, bpost, Wlin, blin,
          ln_g, ln_b):
    return pl.pallas_call(
        functools.partial(_post_kernel, fin),
        grid=(GRID,),
        in_specs=[
            pl.BlockSpec((BLK, fin), lambda i: (i, 0)),
            pl.BlockSpec((BLK, Kp), lambda i: (i, 0)),
            pl.BlockSpec((BLK, Kp), lambda i: (i, 0)),
            pl.BlockSpec((BLK, Kp), lambda i: (i, 0)),
            pl.BlockSpec((BLK, Kp), lambda i: (i, 0)),
            pl.BlockSpec((BLK, Kp), lambda i: (i, 0)),
            pl.BlockSpec((BLK, 16), lambda i: (i, 0)),
            pl.BlockSpec((TOWERS, 13 * fin, F_OUT_T), lambda i: (0, 0, 0)),
            pl.BlockSpec((TOWERS, F_OUT_T), lambda i: (0, 0)),
            pl.BlockSpec((HIDDEN, HIDDEN), lambda i: (0, 0)),
            pl.BlockSpec((HIDDEN,), lambda i: (0,)),
            pl.BlockSpec((HIDDEN,), lambda i: (0,)),
            pl.BlockSpec((HIDDEN,), lambda i: (0,)),
        ],
        out_specs=pl.BlockSpec((BLK, HIDDEN), lambda i: (i, 0)),
        out_shape=jax.ShapeDtypeStruct((N, HIDDEN), jnp.float32),
    )(x, A, S, Q, MN, MX, count, Wpost, bpost, Wlin, blin, ln_g, ln_b)


def _pool_kernel(xb, batch_ref, g_out):
    i = pl.program_id(0)

    @pl.when(i == 0)
    def _():
        g_out[...] = jnp.zeros_like(g_out)

    b = batch_ref[...].reshape(1, BLK)
    onehot = (jax.lax.broadcasted_iota(jnp.int32, (NG, BLK), 0)
              == b).astype(jnp.float32)
    g_out[...] += onehot @ xb[...]


def _pool(x, batch):
    return pl.pallas_call(
        _pool_kernel,
        grid=(GRID,),
        in_specs=[
            pl.BlockSpec((BLK, HIDDEN), lambda i: (i, 0)),
            pl.BlockSpec((1, 1, BLK), lambda i: (i, 0, 0)),
        ],
        out_specs=pl.BlockSpec((NG, HIDDEN), lambda i: (0, 0)),
        out_shape=jax.ShapeDtypeStruct((NG, HIDDEN), jnp.float32),
    )(x, batch.reshape(GRID, 1, BLK))


def _head_kernel(g_ref, l1w, l1b, ln1g, ln1b, l2w, l2b, ln2g, ln2b,
                 o1w, o1b, o2w, o2b, o3w, o3b, out_ref):
    g = g_ref[...]
    g = jax.nn.relu(_layer_norm(g @ l1w[...] + l1b[...], ln1g[...], ln1b[...]))
    g = jax.nn.relu(_layer_norm(g @ l2w[...] + l2b[...], ln2g[...], ln2b[...]))
    g = jax.nn.relu(g @ o1w[...] + o1b[...])
    g = jax.nn.relu(g @ o2w[...] + o2b[...])
    g = g @ o3w[...] + o3b[...]
    out_ref[...] = jnp.abs(g)


def _head(g, m, o):
    args = (g, m["l1"]["W"], m["l1"]["b"], m["ln1_g"], m["ln1_b"],
            m["l2"]["W"], m["l2"]["b"], m["ln2_g"], m["ln2_b"],
            o["o1"]["W"], o["o1"]["b"], o["o2"]["W"], o["o2"]["b"],
            o["o3"]["W"], o["o3"]["b"])
    return pl.pallas_call(
        _head_kernel,
        out_shape=jax.ShapeDtypeStruct((NG, 3), jnp.float32),
    )(*args)


def _conv_layer(c, x, sorted_fields, hist, count):
    fin = x.shape[1]
    K = TOWERS * fin
    Kp = 128 if K == 36 else K

    Wpre = jnp.concatenate([c["pre"][t]["W"] for t in range(TOWERS)], axis=1)
    bpre = jnp.concatenate([c["pre"][t]["b"] for t in range(TOWERS)], axis=0)
    if Kp != K:
        Wpre = jnp.pad(Wpre, ((0, 0), (0, Kp - K)))
        bpre = jnp.pad(bpre, (0, Kp - K))
    Wd, Ws, We2 = Wpre[:fin], Wpre[fin:2 * fin], Wpre[2 * fin:]

    A, B, M3 = _proj(x, Wd, Ws, We2, c["edge"]["b"], bpre, c["edge"]["W"])

    srcs, dsts, a0s, a1s, a2s = sorted_fields
    accum = _make_sc_accum(Kp)
    S, Q, MN, MX, cnt16 = accum(srcs, dsts, a0s, a1s, a2s, hist, B, M3)
    if count is None:
        count = cnt16

    Wpost = jnp.stack([c["post"][t]["W"] for t in range(TOWERS)])
    bpost = jnp.stack([c["post"][t]["b"] for t in range(TOWERS)])
    out = _post(fin, Kp, x, A, S, Q, MN, MX, count, Wpost, bpost,
                c["lin"]["W"], c["lin"]["b"], c["ln_g"], c["ln_b"])
    return out, count


def kernel(x, edge_attr, params, edge_index, batch):
    src = edge_index[0].astype(jnp.int32)
    dst = edge_index[1].astype(jnp.int32)
    pad = E_IN_PAD - E
    src_p = jnp.pad(src, (0, pad))
    dst_p = jnp.pad(dst, (0, pad))
    attr_p = jnp.pad(edge_attr.astype(jnp.float32), ((0, pad), (0, 0))).T

    hist = _sc_hist_k()(dst_p)
    sorted_fields = _sc_permute_k()(src_p, dst_p, attr_p[0], attr_p[1],
                                    attr_p[2], hist)

    count = None
    for c in params["convs"]:
        x, count = _conv_layer(c, x, sorted_fields, hist, count)

    g = _pool(x, batch.astype(jnp.int32))
    return _head(g, params["mlp"], params["out"])


# speedup vs baseline: 4.8422x; 1.0386x over previous
import functools

import jax
import jax.numpy as jnp
import numpy as np
from jax import lax
from jax.experimental import pallas as pl
from jax.experimental.pallas import tpu as pltpu
from jax.experimental.pallas import tpu_sc as plsc

N = 10000
E = 160000
NG = 128
HIDDEN = 64
TOWERS = 4
F_OUT_T = HIDDEN // TOWERS
AVG_LOG = float(np.log(17.0))
BLK = 2000
GRID = N // BLK

NTILES = 32
BSHIFT = 5
BW = 1 << BSHIFT
NB = 320
N_PAD = NB * BW
NBPT = NB // NTILES
CH = 5024
E_IN_PAD = NTILES * CH
C3 = 256
E_PAD = E + 264 + 128
DUMP = E + 264

def _wid():
    return lax.axis_index("s") * 2 + lax.axis_index("c")


@functools.cache
def _sc_hist_k():
    return functools.partial(
        pl.kernel,
        out_type=jax.ShapeDtypeStruct((NTILES, NB), jnp.int32),
        mesh=plsc.VectorSubcoreMesh(core_axis_name="c", subcore_axis_name="s"),
        scratch_types=[
            pltpu.VMEM((CH + 16,), jnp.int32),
            pltpu.VMEM((NB,), jnp.int32),
            pltpu.SMEM((NB,), jnp.int32),
        ],
    )(_sc_hist_body)


def _sc_hist_body(dst_hbm, hist_out, dstv, histv, histm):
    w = _wid()
    base = w * CH
    cnt = jnp.minimum(CH, E - base)
    pltpu.sync_copy(dst_hbm.at[pl.ds(base, CH)], dstv.at[pl.ds(0, CH)])

    def z(b, c):
        histm[b] = 0
        return c

    lax.fori_loop(0, NB, z, 0)

    def body(i, carry):
        b = lax.shift_right_logical(dstv[pl.ds(i, 16)][0], BSHIFT)
        histm[b] = histm[b] + 1
        return carry

    lax.fori_loop(0, cnt, body, 0)

    lane = lax.iota(jnp.int32, 16)
    zero16 = jnp.zeros((16,), jnp.int32)

    def red(bg, c):
        tot = zero16
        for l in range(16):
            tot = jnp.where(lane == l, histm[bg * 16 + l], tot)
        histv[pl.ds(bg * 16, 16)] = tot
        return c

    lax.fori_loop(0, NB // 16, red, 0)
    pltpu.sync_copy(histv, hist_out.at[w])


@functools.cache
def _sc_permute_k():
    return functools.partial(
        pl.kernel,
        out_type=[
            jax.ShapeDtypeStruct((E_PAD,), jnp.int32),
            jax.ShapeDtypeStruct((E_PAD,), jnp.int32),
            jax.ShapeDtypeStruct((E_PAD,), jnp.float32),
            jax.ShapeDtypeStruct((E_PAD,), jnp.float32),
            jax.ShapeDtypeStruct((E_PAD,), jnp.float32),
        ],
        mesh=plsc.VectorSubcoreMesh(core_axis_name="c", subcore_axis_name="s"),
        scratch_types=[
            pltpu.VMEM((NTILES, NB), jnp.int32),
            pltpu.SMEM((NB,), jnp.int32),
            pltpu.VMEM((5040,), jnp.int32),
            pltpu.VMEM((5040,), jnp.int32),
            pltpu.VMEM((5040,), jnp.float32),
            pltpu.VMEM((5040,), jnp.float32),
            pltpu.VMEM((5040,), jnp.float32),
            pltpu.VMEM((45, 112), jnp.int32),
            pltpu.VMEM((256,), jnp.float32),
            pltpu.VMEM((256,), jnp.int32),
            pltpu.SemaphoreType.DMA,
        ],
    )(_sc_permute_body)


def _sc_permute_body(src_hbm, dst_hbm, a0_hbm, a1_hbm, a2_hbm, hist_hbm,
                srcs_out, dsts_out, a0s_out, a1s_out, a2s_out,
                histv, offsm, sv, dv, a0v, a1v, a2v, posv, zbuf, zbi, sem):
    w = _wid()
    base = w * CH
    cnt = jnp.minimum(CH, E - base)
    pltpu.sync_copy(hist_hbm, histv)
    pltpu.sync_copy(src_hbm.at[pl.ds(base, CH)], sv.at[pl.ds(0, CH)])
    pltpu.sync_copy(dst_hbm.at[pl.ds(base, CH)], dv.at[pl.ds(0, CH)])
    pltpu.sync_copy(a0_hbm.at[pl.ds(base, CH)], a0v.at[pl.ds(0, CH)])
    pltpu.sync_copy(a1_hbm.at[pl.ds(base, CH)], a1v.at[pl.ds(0, CH)])
    pltpu.sync_copy(a2_hbm.at[pl.ds(base, CH)], a2v.at[pl.ds(0, CH)])

    lane = lax.iota(jnp.int32, 16)
    zero16 = jnp.zeros((16,), jnp.int32)

    def ob(bg, run):
        tot16 = zero16
        mine16 = zero16

        def iw(w2, c):
            t16, m16 = c
            h16 = histv[w2, pl.ds(bg * 16, 16)]
            return (t16 + h16, m16 + jnp.where(w2 < w, h16, zero16))

        tot16, mine16 = lax.fori_loop(0, NTILES, iw, (tot16, mine16))
        for l in range(16):
            offsm[bg * 16 + l] = run + mine16[l]
            run = run + tot16[l]
        return run

    lax.fori_loop(0, NB // 16, ob, 0)

    dumpv = jnp.full((16,), DUMP, jnp.int32)

    def dump(g, carry):
        cc = g // 7
        gi = g - cc * 7
        posv[cc, pl.ds(gi * 16, 16)] = dumpv
        return carry

    lax.fori_loop(0, 315, dump, 0)

    def grp(g, carry):
        cc = g // 7
        gi = g - cc * 7
        b16 = lax.shift_right_logical(dv[pl.ds(g * 16, 16)], BSHIFT)
        pos16 = zero16
        for l in range(16):
            b = b16[l]
            p = offsm[b]
            offsm[b] = p + 1
            pos16 = jnp.where(lane == l, p, pos16)
        posv[cc, pl.ds(gi * 16, 16)] = pos16
        return carry

    lax.fori_loop(0, lax.shift_right_arithmetic(cnt, 4), grp, 0)

    pending = []
    for c in range(45):
        sl = pl.ds(c * 112, 112)
        idx = posv.at[c]
        pending.append([
            pltpu.async_copy(sv.at[sl], srcs_out.at[idx], sem),
            pltpu.async_copy(dv.at[sl], dsts_out.at[idx], sem),
            pltpu.async_copy(a0v.at[sl], a0s_out.at[idx], sem),
            pltpu.async_copy(a1v.at[sl], a1s_out.at[idx], sem),
            pltpu.async_copy(a2v.at[sl], a2s_out.at[idx], sem),
        ])
        if c >= 8:
            for d in pending[c - 8]:
                d.wait()
    for grp in pending[-8:]:
        for d in grp:
            d.wait()

    @pl.when(w == 0)
    def _():
        for i in range(16):
            zbuf[pl.ds(i * 16, 16)] = jnp.zeros((16,), jnp.float32)
            zbi[pl.ds(i * 16, 16)] = jnp.zeros((16,), jnp.int32)
        pltpu.sync_copy(zbi, srcs_out.at[pl.ds(E, 256)])
        pltpu.sync_copy(zbi, dsts_out.at[pl.ds(E, 256)])
        pltpu.sync_copy(zbuf, a0s_out.at[pl.ds(E, 256)])
        pltpu.sync_copy(zbuf, a1s_out.at[pl.ds(E, 256)])
        pltpu.sync_copy(zbuf, a2s_out.at[pl.ds(E, 256)])


@functools.cache
def _make_sc_accum(K):
    NJ = K // 16

    @functools.partial(
        pl.kernel,
        out_type=[
            jax.ShapeDtypeStruct((N_PAD, K), jnp.float32),
            jax.ShapeDtypeStruct((N_PAD, K), jnp.float32),
            jax.ShapeDtypeStruct((N_PAD, K), jnp.float32),
            jax.ShapeDtypeStruct((N_PAD, K), jnp.float32),
            jax.ShapeDtypeStruct((N_PAD, 16), jnp.float32),
        ],
        mesh=plsc.VectorSubcoreMesh(core_axis_name="c", subcore_axis_name="s"),
        scratch_types=[
            pltpu.VMEM((NTILES, NB), jnp.int32),
            pltpu.SMEM((NB + 1,), jnp.int32),
            pltpu.VMEM((288,), jnp.int32),
            pltpu.VMEM((288,), jnp.int32),
            pltpu.VMEM((288,), jnp.float32),
            pltpu.VMEM((288,), jnp.float32),
            pltpu.VMEM((288,), jnp.float32),
            pltpu.VMEM((272, K), jnp.float32),
            pltpu.VMEM((3, K), jnp.float32),
            pltpu.VMEM((BW, K), jnp.float32),
            pltpu.VMEM((BW, K), jnp.float32),
            pltpu.VMEM((BW, K), jnp.float32),
            pltpu.VMEM((BW, K), jnp.float32),
            pltpu.VMEM((BW, 16), jnp.float32),
            pltpu.SemaphoreType.DMA,
        ],
    )
    def _sc_accum(srcs_hbm, dsts_hbm, a0_hbm, a1_hbm, a2_hbm, hist_hbm,
                  B_hbm, m3_hbm, S_out, Q_out, MN_out, MX_out, cnt_out,
                  histv, prefm, sv, dv, a0v, a1v, a2v, rowsv, m3v,
                  accS, accQ, accMN, accMX, accC, sem):
        w = _wid()
        pltpu.sync_copy(hist_hbm, histv)
        pltpu.sync_copy(m3_hbm, m3v)

        zero16i = jnp.zeros((16,), jnp.int32)

        def pb(bg, run):
            def iw(w2, t16):
                return t16 + histv[w2, pl.ds(bg * 16, 16)]

            tot16 = lax.fori_loop(0, NTILES, iw, zero16i)
            for l in range(16):
                prefm[bg * 16 + l] = run
                run = run + tot16[l]
            return run

        run = lax.fori_loop(0, NB // 16, pb, 0)
        prefm[NB] = run

        zero16 = jnp.zeros((16,), jnp.float32)
        big16 = jnp.full((16,), 1e30, jnp.float32)
        ones16 = jnp.ones((16,), jnp.float32)

        def tbody(t, tcarry):
            b = w + NTILES * t
            start = prefm[b]
            end = prefm[b + 1]

            def zr(r, carry):
                for j in range(NJ):
                    accS[r, pl.ds(j * 16, 16)] = zero16
                    accQ[r, pl.ds(j * 16, 16)] = zero16
                    accMN[r, pl.ds(j * 16, 16)] = big16
                    accMX[r, pl.ds(j * 16, 16)] = -big16
                accC[r, pl.ds(0, 16)] = zero16
                return carry

            lax.fori_loop(0, BW, zr, 0)

            def chunk_body(k, kcarry):
                cbase = start + k * C3
                a8 = pl.multiple_of(lax.bitwise_and(cbase, ~15), 16)
                off = cbase - a8
                valid = jnp.minimum(C3, end - cbase)
                pltpu.sync_copy(srcs_hbm.at[pl.ds(a8, 288)], sv)
                g1 = pltpu.async_copy(B_hbm.at[sv.at[pl.ds(0, 128)]],
                                      rowsv.at[pl.ds(0, 128)], sem)
                g2 = pltpu.async_copy(B_hbm.at[sv.at[pl.ds(128, 128)]],
                                      rowsv.at[pl.ds(128, 128)], sem)
                g3 = pltpu.async_copy(B_hbm.at[sv.at[pl.ds(256, 16)]],
                                      rowsv.at[pl.ds(256, 16)], sem)
                f1 = pltpu.async_copy(dsts_hbm.at[pl.ds(a8, 288)], dv, sem)
                f2 = pltpu.async_copy(a0_hbm.at[pl.ds(a8, 288)], a0v, sem)
                f3 = pltpu.async_copy(a1_hbm.at[pl.ds(a8, 288)], a1v, sem)
                f4 = pltpu.async_copy(a2_hbm.at[pl.ds(a8, 288)], a2v, sem)
                g1.wait()
                g2.wait()
                g3.wait()
                f1.wait()
                f2.wait()
                f3.wait()
                f4.wait()

                def edge(i, carry):
                    ii = off + i
                    dl = lax.bitwise_and(dv[pl.ds(ii, 16)][0], BW - 1)
                    a0 = a0v[pl.ds(ii, 16)][0]
                    a1 = a1v[pl.ds(ii, 16)][0]
                    a2 = a2v[pl.ds(ii, 16)][0]
                    for j in range(NJ):
                        sl = pl.ds(j * 16, 16)
                        m = (rowsv[ii, sl] + a0 * m3v[0, sl]
                             + a1 * m3v[1, sl] + a2 * m3v[2, sl])
                        plsc.addupdate(accS.at[dl, sl], m)
                        plsc.addupdate(accQ.at[dl, sl], m * m)
                        accMN[dl, sl] = jnp.minimum(accMN[dl, sl], m)
                        accMX[dl, sl] = jnp.maximum(accMX[dl, sl], m)
                    plsc.addupdate(accC.at[dl, pl.ds(0, 16)], ones16)
                    return carry

                lax.fori_loop(0, valid, edge, 0)
                return kcarry

            nch = lax.shift_right_arithmetic(end - start + C3 - 1, 8)
            lax.fori_loop(0, nch, chunk_body, 0)

            rsl = pl.ds(b * BW, BW)
            pltpu.sync_copy(accS, S_out.at[rsl])
            pltpu.sync_copy(accQ, Q_out.at[rsl])
            pltpu.sync_copy(accMN, MN_out.at[rsl])
            pltpu.sync_copy(accMX, MX_out.at[rsl])
            pltpu.sync_copy(accC, cnt_out.at[rsl])
            return tcarry

        lax.fori_loop(0, NBPT, tbody, 0)

    return _sc_accum


def _layer_norm(x, g, b):
    mu = jnp.mean(x, axis=-1, keepdims=True)
    var = jnp.mean((x - mu) ** 2, axis=-1, keepdims=True)
    return (x - mu) / jnp.sqrt(var + 1e-5) * g + b


def _proj_kernel(xb, Wd, Ws, We2, be, bpre, We, A_out, B_out, M3_out):
    c0 = be[...] @ We2[...]
    A_out[...] = xb[...] @ Wd[...] + bpre[...] + c0
    B_out[...] = xb[...] @ Ws[...]
    M3_out[...] = We[...] @ We2[...]


def _proj(x, Wd, Ws, We2, be, bpre, We):
    fin = x.shape[1]
    Kp = Wd.shape[1]
    return pl.pallas_call(
        _proj_kernel,
        grid=(GRID,),
        in_specs=[
            pl.BlockSpec((BLK, fin), lambda i: (i, 0)),
            pl.BlockSpec((fin, Kp), lambda i: (0, 0)),
            pl.BlockSpec((fin, Kp), lambda i: (0, 0)),
            pl.BlockSpec((fin, Kp), lambda i: (0, 0)),
            pl.BlockSpec((fin,), lambda i: (0,)),
            pl.BlockSpec((Kp,), lambda i: (0,)),
            pl.BlockSpec((3, fin), lambda i: (0, 0)),
        ],
        out_specs=[
            pl.BlockSpec((BLK, Kp), lambda i: (i, 0)),
            pl.BlockSpec((BLK, Kp), lambda i: (i, 0)),
            pl.BlockSpec((3, Kp), lambda i: (0, 0)),
        ],
        out_shape=[
            jax.ShapeDtypeStruct((N, Kp), jnp.float32),
            jax.ShapeDtypeStruct((N, Kp), jnp.float32),
            jax.ShapeDtypeStruct((3, Kp), jnp.float32),
        ],
    )(x, Wd, Ws, We2, be, bpre, We)


def _post_kernel(fin, xb, Ab, Sb, Qb, MNb, MXb, count_ref, Wpost, bpost,
                 Wlin, blin, ln_g, ln_b, out_ref):
    K = 4 * fin
    cntf = count_ref[...][:, 0:1]
    has = cntf > 0.0
    cnt = jnp.maximum(cntf, 1.0)
    logdeg = jnp.log(cnt + 1.0)
    amp = logdeg / AVG_LOG
    att = AVG_LOG / logdeg

    A = Ab[...]
    S = Sb[...]
    Q = Qb[...]
    mean = jnp.where(has, A + S / cnt, 0.0)
    var = jnp.maximum(Q / cnt - (S / cnt) ** 2, 0.0)
    std = jnp.sqrt(jnp.where(has, var, 0.0) + 1e-5)
    mn = jnp.where(has, A + MNb[...], 0.0)
    mx = jnp.where(has, A + MXb[...], 0.0)

    y = jnp.zeros((BLK, HIDDEN), jnp.float32)
    bias = blin[...]
    stats = (mean, mn, mx, std)
    for t in range(TOWERS):
        Wl_t = Wlin[t * F_OUT_T:(t + 1) * F_OUT_T, :]
        Wp = Wpost[t]
        Px_t = Wp[0:fin, :] @ Wl_t
        y += xb[...] @ Px_t
        bias += bpost[t] @ Wl_t
        for g in range(3):
            scale = (1.0, amp, att)[g]
            acc = jnp.zeros((BLK, HIDDEN), jnp.float32)
            for s in range(4):
                lo = fin + g * K + s * fin
                Wrows = Wp[lo:lo + fin, :] @ Wl_t
                acc += stats[s][:, t * fin:(t + 1) * fin] @ Wrows
            y = y + acc * scale
    y = y + bias
    out_ref[...] = jax.nn.relu(_layer_norm(y, ln_g[...], ln_b[...]))


def _post(fin, Kp, x, A, S, Q, MN, MX, count, Wpost, bpost, Wlin, blin,
          ln_g, ln_b):
    return pl.pallas_call(
        functools.partial(_post_kernel, fin),
        grid=(GRID,),
        in_specs=[
            pl.BlockSpec((BLK, fin), lambda i: (i, 0)),
            pl.BlockSpec((BLK, Kp), lambda i: (i, 0)),
            pl.BlockSpec((BLK, Kp), lambda i: (i, 0)),
            pl.BlockSpec((BLK, Kp), lambda i: (i, 0)),
            pl.BlockSpec((BLK, Kp), lambda i: (i, 0)),
            pl.BlockSpec((BLK, Kp), lambda i: (i, 0)),
            pl.BlockSpec((BLK, 16), lambda i: (i, 0)),
            pl.BlockSpec((TOWERS, 13 * fin, F_OUT_T), lambda i: (0, 0, 0)),
            pl.BlockSpec((TOWERS, F_OUT_T), lambda i: (0, 0)),
            pl.BlockSpec((HIDDEN, HIDDEN), lambda i: (0, 0)),
            pl.BlockSpec((HIDDEN,), lambda i: (0,)),
            pl.BlockSpec((HIDDEN,), lambda i: (0,)),
            pl.BlockSpec((HIDDEN,), lambda i: (0,)),
        ],
        out_specs=pl.BlockSpec((BLK, HIDDEN), lambda i: (i, 0)),
        out_shape=jax.ShapeDtypeStruct((N, HIDDEN), jnp.float32),
    )(x, A, S, Q, MN, MX, count, Wpost, bpost, Wlin, blin, ln_g, ln_b)


def _pool_kernel(xb, batch_ref, g_out):
    i = pl.program_id(0)

    @pl.when(i == 0)
    def _():
        g_out[...] = jnp.zeros_like(g_out)

    b = batch_ref[...].reshape(1, BLK)
    onehot = (jax.lax.broadcasted_iota(jnp.int32, (NG, BLK), 0)
              == b).astype(jnp.float32)
    g_out[...] += onehot @ xb[...]


def _pool(x, batch):
    return pl.pallas_call(
        _pool_kernel,
        grid=(GRID,),
        in_specs=[
            pl.BlockSpec((BLK, HIDDEN), lambda i: (i, 0)),
            pl.BlockSpec((1, 1, BLK), lambda i: (i, 0, 0)),
        ],
        out_specs=pl.BlockSpec((NG, HIDDEN), lambda i: (0, 0)),
        out_shape=jax.ShapeDtypeStruct((NG, HIDDEN), jnp.float32),
    )(x, batch.reshape(GRID, 1, BLK))


def _head_kernel(g_ref, l1w, l1b, ln1g, ln1b, l2w, l2b, ln2g, ln2b,
                 o1w, o1b, o2w, o2b, o3w, o3b, out_ref):
    g = g_ref[...]
    g = jax.nn.relu(_layer_norm(g @ l1w[...] + l1b[...], ln1g[...], ln1b[...]))
    g = jax.nn.relu(_layer_norm(g @ l2w[...] + l2b[...], ln2g[...], ln2b[...]))
    g = jax.nn.relu(g @ o1w[...] + o1b[...])
    g = jax.nn.relu(g @ o2w[...] + o2b[...])
    g = g @ o3w[...] + o3b[...]
    out_ref[...] = jnp.abs(g)


def _head(g, m, o):
    args = (g, m["l1"]["W"], m["l1"]["b"], m["ln1_g"], m["ln1_b"],
            m["l2"]["W"], m["l2"]["b"], m["ln2_g"], m["ln2_b"],
            o["o1"]["W"], o["o1"]["b"], o["o2"]["W"], o["o2"]["b"],
            o["o3"]["W"], o["o3"]["b"])
    return pl.pallas_call(
        _head_kernel,
        out_shape=jax.ShapeDtypeStruct((NG, 3), jnp.float32),
    )(*args)


def _conv_layer(c, x, sorted_fields, hist, count):
    fin = x.shape[1]
    K = TOWERS * fin
    Kp = 128 if K == 36 else K

    Wpre = jnp.concatenate([c["pre"][t]["W"] for t in range(TOWERS)], axis=1)
    bpre = jnp.concatenate([c["pre"][t]["b"] for t in range(TOWERS)], axis=0)
    if Kp != K:
        Wpre = jnp.pad(Wpre, ((0, 0), (0, Kp - K)))
        bpre = jnp.pad(bpre, (0, Kp - K))
    Wd, Ws, We2 = Wpre[:fin], Wpre[fin:2 * fin], Wpre[2 * fin:]

    A, B, M3 = _proj(x, Wd, Ws, We2, c["edge"]["b"], bpre, c["edge"]["W"])

    srcs, dsts, a0s, a1s, a2s = sorted_fields
    accum = _make_sc_accum(Kp)
    S, Q, MN, MX, cnt16 = accum(srcs, dsts, a0s, a1s, a2s, hist, B, M3)
    if count is None:
        count = cnt16

    Wpost = jnp.stack([c["post"][t]["W"] for t in range(TOWERS)])
    bpost = jnp.stack([c["post"][t]["b"] for t in range(TOWERS)])
    out = _post(fin, Kp, x, A, S, Q, MN, MX, count, Wpost, bpost,
                c["lin"]["W"], c["lin"]["b"], c["ln_g"], c["ln_b"])
    return out, count


def kernel(x, edge_attr, params, edge_index, batch):
    src = edge_index[0].astype(jnp.int32)
    dst = edge_index[1].astype(jnp.int32)
    pad = E_IN_PAD - E
    src_p = jnp.pad(src, (0, pad))
    dst_p = jnp.pad(dst, (0, pad))
    attr_p = jnp.pad(edge_attr.astype(jnp.float32), ((0, pad), (0, 0))).T

    hist = _sc_hist_k()(dst_p)
    sorted_fields = _sc_permute_k()(src_p, dst_p, attr_p[0], attr_p[1],
                                    attr_p[2], hist)

    count = None
    for c in params["convs"]:
        x, count = _conv_layer(c, x, sorted_fields, hist, count)

    g = _pool(x, batch.astype(jnp.int32))
    return _head(g, params["mlp"], params["out"])


# speedup vs baseline: 5.0909x; 1.0514x over previous
import functools

import jax
import jax.numpy as jnp
import numpy as np
from jax import lax
from jax.experimental import pallas as pl
from jax.experimental.pallas import tpu as pltpu
from jax.experimental.pallas import tpu_sc as plsc

N = 10000
E = 160000
NG = 128
HIDDEN = 64
TOWERS = 4
F_OUT_T = HIDDEN // TOWERS
AVG_LOG = float(np.log(17.0))
BLK = 2000
GRID = N // BLK

NTILES = 32
BSHIFT = 5
BW = 1 << BSHIFT
NB = 320
N_PAD = NB * BW
NBPT = NB // NTILES
CH = 5024
E_IN_PAD = NTILES * CH
C3 = 256
E_PAD = E + 264 + 128
DUMP = E + 264

def _wid():
    return lax.axis_index("s") * 2 + lax.axis_index("c")


@functools.cache
def _sc_hist_k():
    return functools.partial(
        pl.kernel,
        out_type=jax.ShapeDtypeStruct((NTILES, NB), jnp.int32),
        mesh=plsc.VectorSubcoreMesh(core_axis_name="c", subcore_axis_name="s"),
        scratch_types=[
            pltpu.VMEM((CH + 16,), jnp.int32),
            pltpu.VMEM((NB,), jnp.int32),
            pltpu.SMEM((NB,), jnp.int32),
        ],
    )(_sc_hist_body)


def _sc_hist_body(dst_hbm, hist_out, dstv, histv, histm):
    w = _wid()
    base = w * CH
    cnt = jnp.minimum(CH, E - base)
    pltpu.sync_copy(dst_hbm.at[pl.ds(base, CH)], dstv.at[pl.ds(0, CH)])

    def z(b, c):
        histm[b] = 0
        return c

    lax.fori_loop(0, NB, z, 0)

    def body(i, carry):
        b = lax.shift_right_logical(dstv[pl.ds(i, 16)][0], BSHIFT)
        histm[b] = histm[b] + 1
        return carry

    lax.fori_loop(0, cnt, body, 0)

    lane = lax.iota(jnp.int32, 16)
    zero16 = jnp.zeros((16,), jnp.int32)

    def red(bg, c):
        tot = zero16
        for l in range(16):
            tot = jnp.where(lane == l, histm[bg * 16 + l], tot)
        histv[pl.ds(bg * 16, 16)] = tot
        return c

    lax.fori_loop(0, NB // 16, red, 0)
    pltpu.sync_copy(histv, hist_out.at[w])


@functools.cache
def _sc_permute_k():
    return functools.partial(
        pl.kernel,
        out_type=[
            jax.ShapeDtypeStruct((E_PAD,), jnp.int32),
            jax.ShapeDtypeStruct((E_PAD,), jnp.int32),
            jax.ShapeDtypeStruct((E_PAD,), jnp.float32),
            jax.ShapeDtypeStruct((E_PAD,), jnp.float32),
            jax.ShapeDtypeStruct((E_PAD,), jnp.float32),
        ],
        mesh=plsc.VectorSubcoreMesh(core_axis_name="c", subcore_axis_name="s"),
        scratch_types=[
            pltpu.VMEM((NTILES, NB), jnp.int32),
            pltpu.SMEM((NB,), jnp.int32),
            pltpu.VMEM((5040,), jnp.int32),
            pltpu.VMEM((5040,), jnp.int32),
            pltpu.VMEM((5040,), jnp.float32),
            pltpu.VMEM((5040,), jnp.float32),
            pltpu.VMEM((5040,), jnp.float32),
            pltpu.VMEM((45, 112), jnp.int32),
            pltpu.VMEM((256,), jnp.float32),
            pltpu.VMEM((256,), jnp.int32),
            pltpu.SemaphoreType.DMA,
        ],
    )(_sc_permute_body)


def _sc_permute_body(src_hbm, dst_hbm, a0_hbm, a1_hbm, a2_hbm, hist_hbm,
                srcs_out, dsts_out, a0s_out, a1s_out, a2s_out,
                histv, offsm, sv, dv, a0v, a1v, a2v, posv, zbuf, zbi, sem):
    w = _wid()
    base = w * CH
    cnt = jnp.minimum(CH, E - base)
    pltpu.sync_copy(hist_hbm, histv)
    pltpu.sync_copy(src_hbm.at[pl.ds(base, CH)], sv.at[pl.ds(0, CH)])
    pltpu.sync_copy(dst_hbm.at[pl.ds(base, CH)], dv.at[pl.ds(0, CH)])
    pltpu.sync_copy(a0_hbm.at[pl.ds(base, CH)], a0v.at[pl.ds(0, CH)])
    pltpu.sync_copy(a1_hbm.at[pl.ds(base, CH)], a1v.at[pl.ds(0, CH)])
    pltpu.sync_copy(a2_hbm.at[pl.ds(base, CH)], a2v.at[pl.ds(0, CH)])

    lane = lax.iota(jnp.int32, 16)
    zero16 = jnp.zeros((16,), jnp.int32)

    def ob(bg, run):
        tot16 = zero16
        mine16 = zero16

        def iw(w2, c):
            t16, m16 = c
            h16 = histv[w2, pl.ds(bg * 16, 16)]
            return (t16 + h16, m16 + jnp.where(w2 < w, h16, zero16))

        tot16, mine16 = lax.fori_loop(0, NTILES, iw, (tot16, mine16))
        for l in range(16):
            offsm[bg * 16 + l] = run + mine16[l]
            run = run + tot16[l]
        return run

    lax.fori_loop(0, NB // 16, ob, 0)

    dumpv = jnp.full((16,), DUMP, jnp.int32)

    def dump(g, carry):
        cc = g // 7
        gi = g - cc * 7
        posv[cc, pl.ds(gi * 16, 16)] = dumpv
        return carry

    lax.fori_loop(0, 315, dump, 0)

    def grp(g, carry):
        cc = g // 7
        gi = g - cc * 7
        b16 = lax.shift_right_logical(dv[pl.ds(g * 16, 16)], BSHIFT)
        pos16 = zero16
        for l in range(16):
            b = b16[l]
            p = offsm[b]
            offsm[b] = p + 1
            pos16 = jnp.where(lane == l, p, pos16)
        posv[cc, pl.ds(gi * 16, 16)] = pos16
        return carry

    lax.fori_loop(0, lax.shift_right_arithmetic(cnt, 4), grp, 0)

    pending = []
    for c in range(45):
        sl = pl.ds(c * 112, 112)
        idx = posv.at[c]
        pending.append([
            pltpu.async_copy(sv.at[sl], srcs_out.at[idx], sem),
            pltpu.async_copy(dv.at[sl], dsts_out.at[idx], sem),
            pltpu.async_copy(a0v.at[sl], a0s_out.at[idx], sem),
            pltpu.async_copy(a1v.at[sl], a1s_out.at[idx], sem),
            pltpu.async_copy(a2v.at[sl], a2s_out.at[idx], sem),
        ])
        if c >= 8:
            for d in pending[c - 8]:
                d.wait()
    for grp in pending[-8:]:
        for d in grp:
            d.wait()

    @pl.when(w == 0)
    def _():
        for i in range(16):
            zbuf[pl.ds(i * 16, 16)] = jnp.zeros((16,), jnp.float32)
            zbi[pl.ds(i * 16, 16)] = jnp.zeros((16,), jnp.int32)
        pltpu.sync_copy(zbi, srcs_out.at[pl.ds(E, 256)])
        pltpu.sync_copy(zbi, dsts_out.at[pl.ds(E, 256)])
        pltpu.sync_copy(zbuf, a0s_out.at[pl.ds(E, 256)])
        pltpu.sync_copy(zbuf, a1s_out.at[pl.ds(E, 256)])
        pltpu.sync_copy(zbuf, a2s_out.at[pl.ds(E, 256)])


@functools.cache
def _make_sc_accum(K):
    NJ = K // 16

    @functools.partial(
        pl.kernel,
        out_type=[
            jax.ShapeDtypeStruct((N_PAD, K), jnp.float32),
            jax.ShapeDtypeStruct((N_PAD, K), jnp.float32),
            jax.ShapeDtypeStruct((N_PAD, K), jnp.float32),
            jax.ShapeDtypeStruct((N_PAD, K), jnp.float32),
            jax.ShapeDtypeStruct((N_PAD, 16), jnp.float32),
        ],
        mesh=plsc.VectorSubcoreMesh(core_axis_name="c", subcore_axis_name="s"),
        scratch_types=[
            pltpu.VMEM((NTILES, NB), jnp.int32),
            pltpu.SMEM((NB + 1,), jnp.int32),
            pltpu.VMEM((288,), jnp.int32),
            pltpu.VMEM((288,), jnp.int32),
            pltpu.VMEM((288,), jnp.float32),
            pltpu.VMEM((288,), jnp.float32),
            pltpu.VMEM((288,), jnp.float32),
            pltpu.VMEM((272, K), jnp.float32),
            pltpu.VMEM((3, K), jnp.float32),
            pltpu.VMEM((BW, K), jnp.float32),
            pltpu.VMEM((BW, K), jnp.float32),
            pltpu.VMEM((BW, K), jnp.float32),
            pltpu.VMEM((BW, K), jnp.float32),
            pltpu.VMEM((BW, 16), jnp.float32),
            pltpu.SemaphoreType.DMA,
        ],
    )
    def _sc_accum(srcs_hbm, dsts_hbm, a0_hbm, a1_hbm, a2_hbm, hist_hbm,
                  B_hbm, m3_hbm, S_out, Q_out, MN_out, MX_out, cnt_out,
                  histv, prefm, sv, dv, a0v, a1v, a2v, rowsv, m3v,
                  accS, accQ, accMN, accMX, accC, sem):
        w = _wid()
        pltpu.sync_copy(hist_hbm, histv)
        pltpu.sync_copy(m3_hbm, m3v)

        zero16i = jnp.zeros((16,), jnp.int32)

        def pb(bg, run):
            def iw(w2, t16):
                return t16 + histv[w2, pl.ds(bg * 16, 16)]

            tot16 = lax.fori_loop(0, NTILES, iw, zero16i)
            for l in range(16):
                prefm[bg * 16 + l] = run
                run = run + tot16[l]
            return run

        run = lax.fori_loop(0, NB // 16, pb, 0)
        prefm[NB] = run

        zero16 = jnp.zeros((16,), jnp.float32)
        big16 = jnp.full((16,), 1e30, jnp.float32)
        ones16 = jnp.ones((16,), jnp.float32)
        m3c = [[m3v[r, pl.ds(j * 16, 16)] for r in range(3)]
               for j in range(NJ)]

        def tbody(t, tcarry):
            b = w + NTILES * t
            start = prefm[b]
            end = prefm[b + 1]

            def zr(r, carry):
                for j in range(NJ):
                    accS[r, pl.ds(j * 16, 16)] = zero16
                    accQ[r, pl.ds(j * 16, 16)] = zero16
                    accMN[r, pl.ds(j * 16, 16)] = big16
                    accMX[r, pl.ds(j * 16, 16)] = -big16
                accC[r, pl.ds(0, 16)] = zero16
                return carry

            lax.fori_loop(0, BW, zr, 0)

            def chunk_body(k, kcarry):
                cbase = start + k * C3
                a8 = pl.multiple_of(lax.bitwise_and(cbase, ~15), 16)
                off = cbase - a8
                valid = jnp.minimum(C3, end - cbase)
                pltpu.sync_copy(srcs_hbm.at[pl.ds(a8, 288)], sv)
                g1 = pltpu.async_copy(B_hbm.at[sv.at[pl.ds(0, 128)]],
                                      rowsv.at[pl.ds(0, 128)], sem)
                g2 = pltpu.async_copy(B_hbm.at[sv.at[pl.ds(128, 128)]],
                                      rowsv.at[pl.ds(128, 128)], sem)
                g3 = pltpu.async_copy(B_hbm.at[sv.at[pl.ds(256, 16)]],
                                      rowsv.at[pl.ds(256, 16)], sem)
                f1 = pltpu.async_copy(dsts_hbm.at[pl.ds(a8, 288)], dv, sem)
                f2 = pltpu.async_copy(a0_hbm.at[pl.ds(a8, 288)], a0v, sem)
                f3 = pltpu.async_copy(a1_hbm.at[pl.ds(a8, 288)], a1v, sem)
                f4 = pltpu.async_copy(a2_hbm.at[pl.ds(a8, 288)], a2v, sem)
                g1.wait()
                g2.wait()
                g3.wait()
                f1.wait()
                f2.wait()
                f3.wait()
                f4.wait()

                def edge(i, carry):
                    ii = off + i
                    dl = lax.bitwise_and(dv[pl.ds(ii, 16)][0], BW - 1)
                    a0 = a0v[pl.ds(ii, 16)][0]
                    a1 = a1v[pl.ds(ii, 16)][0]
                    a2 = a2v[pl.ds(ii, 16)][0]
                    for j in range(NJ):
                        sl = pl.ds(j * 16, 16)
                        m = (rowsv[ii, sl] + a0 * m3c[j][0]
                             + a1 * m3c[j][1] + a2 * m3c[j][2])
                        plsc.addupdate(accS.at[dl, sl], m)
                        plsc.addupdate(accQ.at[dl, sl], m * m)
                        accMN[dl, sl] = jnp.minimum(accMN[dl, sl], m)
                        accMX[dl, sl] = jnp.maximum(accMX[dl, sl], m)
                    plsc.addupdate(accC.at[dl, pl.ds(0, 16)], ones16)
                    return carry

                lax.fori_loop(0, valid, edge, 0)
                return kcarry

            nch = lax.shift_right_arithmetic(end - start + C3 - 1, 8)
            lax.fori_loop(0, nch, chunk_body, 0)

            rsl = pl.ds(b * BW, BW)
            pltpu.sync_copy(accS, S_out.at[rsl])
            pltpu.sync_copy(accQ, Q_out.at[rsl])
            pltpu.sync_copy(accMN, MN_out.at[rsl])
            pltpu.sync_copy(accMX, MX_out.at[rsl])
            pltpu.sync_copy(accC, cnt_out.at[rsl])
            return tcarry

        lax.fori_loop(0, NBPT, tbody, 0)

    return _sc_accum


def _layer_norm(x, g, b):
    mu = jnp.mean(x, axis=-1, keepdims=True)
    var = jnp.mean((x - mu) ** 2, axis=-1, keepdims=True)
    return (x - mu) / jnp.sqrt(var + 1e-5) * g + b


def _proj_kernel(xb, Wd, Ws, We2, be, bpre, We, A_out, B_out, M3_out):
    c0 = be[...] @ We2[...]
    A_out[...] = xb[...] @ Wd[...] + bpre[...] + c0
    B_out[...] = xb[...] @ Ws[...]
    M3_out[...] = We[...] @ We2[...]


def _proj(x, Wd, Ws, We2, be, bpre, We):
    fin = x.shape[1]
    Kp = Wd.shape[1]
    return pl.pallas_call(
        _proj_kernel,
        grid=(GRID,),
        in_specs=[
            pl.BlockSpec((BLK, fin), lambda i: (i, 0)),
            pl.BlockSpec((fin, Kp), lambda i: (0, 0)),
            pl.BlockSpec((fin, Kp), lambda i: (0, 0)),
            pl.BlockSpec((fin, Kp), lambda i: (0, 0)),
            pl.BlockSpec((fin,), lambda i: (0,)),
            pl.BlockSpec((Kp,), lambda i: (0,)),
            pl.BlockSpec((3, fin), lambda i: (0, 0)),
        ],
        out_specs=[
            pl.BlockSpec((BLK, Kp), lambda i: (i, 0)),
            pl.BlockSpec((BLK, Kp), lambda i: (i, 0)),
            pl.BlockSpec((3, Kp), lambda i: (0, 0)),
        ],
        out_shape=[
            jax.ShapeDtypeStruct((N, Kp), jnp.float32),
            jax.ShapeDtypeStruct((N, Kp), jnp.float32),
            jax.ShapeDtypeStruct((3, Kp), jnp.float32),
        ],
    )(x, Wd, Ws, We2, be, bpre, We)


def _post_kernel(fin, xb, Ab, Sb, Qb, MNb, MXb, count_ref, Wpost, bpost,
                 Wlin, blin, ln_g, ln_b, out_ref):
    K = 4 * fin
    cntf = count_ref[...][:, 0:1]
    has = cntf > 0.0
    cnt = jnp.maximum(cntf, 1.0)
    logdeg = jnp.log(cnt + 1.0)
    amp = logdeg / AVG_LOG
    att = AVG_LOG / logdeg

    A = Ab[...]
    S = Sb[...]
    Q = Qb[...]
    mean = jnp.where(has, A + S / cnt, 0.0)
    var = jnp.maximum(Q / cnt - (S / cnt) ** 2, 0.0)
    std = jnp.sqrt(jnp.where(has, var, 0.0) + 1e-5)
    mn = jnp.where(has, A + MNb[...], 0.0)
    mx = jnp.where(has, A + MXb[...], 0.0)

    y = jnp.zeros((BLK, HIDDEN), jnp.float32)
    bias = blin[...]
    stats = (mean, mn, mx, std)
    for t in range(TOWERS):
        Wl_t = Wlin[t * F_OUT_T:(t + 1) * F_OUT_T, :]
        Wp = Wpost[t]
        Px_t = Wp[0:fin, :] @ Wl_t
        y += xb[...] @ Px_t
        bias += bpost[t] @ Wl_t
        for g in range(3):
            scale = (1.0, amp, att)[g]
            acc = jnp.zeros((BLK, HIDDEN), jnp.float32)
            for s in range(4):
                lo = fin + g * K + s * fin
                Wrows = Wp[lo:lo + fin, :] @ Wl_t
                acc += stats[s][:, t * fin:(t + 1) * fin] @ Wrows
            y = y + acc * scale
    y = y + bias
    out_ref[...] = jax.nn.relu(_layer_norm(y, ln_g[...], ln_b[...]))


def _post(fin, Kp, x, A, S, Q, MN, MX, count, Wpost, bpost, Wlin, blin,
          ln_g, ln_b):
    return pl.pallas_call(
        functools.partial(_post_kernel, fin),
        grid=(GRID,),
        in_specs=[
            pl.BlockSpec((BLK, fin), lambda i: (i, 0)),
            pl.BlockSpec((BLK, Kp), lambda i: (i, 0)),
            pl.BlockSpec((BLK, Kp), lambda i: (i, 0)),
            pl.BlockSpec((BLK, Kp), lambda i: (i, 0)),
            pl.BlockSpec((BLK, Kp), lambda i: (i, 0)),
            pl.BlockSpec((BLK, Kp), lambda i: (i, 0)),
            pl.BlockSpec((BLK, 16), lambda i: (i, 0)),
            pl.BlockSpec((TOWERS, 13 * fin, F_OUT_T), lambda i: (0, 0, 0)),
            pl.BlockSpec((TOWERS, F_OUT_T), lambda i: (0, 0)),
            pl.BlockSpec((HIDDEN, HIDDEN), lambda i: (0, 0)),
            pl.BlockSpec((HIDDEN,), lambda i: (0,)),
            pl.BlockSpec((HIDDEN,), lambda i: (0,)),
            pl.BlockSpec((HIDDEN,), lambda i: (0,)),
        ],
        out_specs=pl.BlockSpec((BLK, HIDDEN), lambda i: (i, 0)),
        out_shape=jax.ShapeDtypeStruct((N, HIDDEN), jnp.float32),
    )(x, A, S, Q, MN, MX, count, Wpost, bpost, Wlin, blin, ln_g, ln_b)


def _pool_kernel(xb, batch_ref, g_out):
    i = pl.program_id(0)

    @pl.when(i == 0)
    def _():
        g_out[...] = jnp.zeros_like(g_out)

    b = batch_ref[...].reshape(1, BLK)
    onehot = (jax.lax.broadcasted_iota(jnp.int32, (NG, BLK), 0)
              == b).astype(jnp.float32)
    g_out[...] += onehot @ xb[...]


def _pool(x, batch):
    return pl.pallas_call(
        _pool_kernel,
        grid=(GRID,),
        in_specs=[
            pl.BlockSpec((BLK, HIDDEN), lambda i: (i, 0)),
            pl.BlockSpec((1, 1, BLK), lambda i: (i, 0, 0)),
        ],
        out_specs=pl.BlockSpec((NG, HIDDEN), lambda i: (0, 0)),
        out_shape=jax.ShapeDtypeStruct((NG, HIDDEN), jnp.float32),
    )(x, batch.reshape(GRID, 1, BLK))


def _head_kernel(g_ref, l1w, l1b, ln1g, ln1b, l2w, l2b, ln2g, ln2b,
                 o1w, o1b, o2w, o2b, o3w, o3b, out_ref):
    g = g_ref[...]
    g = jax.nn.relu(_layer_norm(g @ l1w[...] + l1b[...], ln1g[...], ln1b[...]))
    g = jax.nn.relu(_layer_norm(g @ l2w[...] + l2b[...], ln2g[...], ln2b[...]))
    g = jax.nn.relu(g @ o1w[...] + o1b[...])
    g = jax.nn.relu(g @ o2w[...] + o2b[...])
    g = g @ o3w[...] + o3b[...]
    out_ref[...] = jnp.abs(g)


def _head(g, m, o):
    args = (g, m["l1"]["W"], m["l1"]["b"], m["ln1_g"], m["ln1_b"],
            m["l2"]["W"], m["l2"]["b"], m["ln2_g"], m["ln2_b"],
            o["o1"]["W"], o["o1"]["b"], o["o2"]["W"], o["o2"]["b"],
            o["o3"]["W"], o["o3"]["b"])
    return pl.pallas_call(
        _head_kernel,
        out_shape=jax.ShapeDtypeStruct((NG, 3), jnp.float32),
    )(*args)


def _conv_layer(c, x, sorted_fields, hist, count):
    fin = x.shape[1]
    K = TOWERS * fin
    Kp = 128 if K == 36 else K

    Wpre = jnp.concatenate([c["pre"][t]["W"] for t in range(TOWERS)], axis=1)
    bpre = jnp.concatenate([c["pre"][t]["b"] for t in range(TOWERS)], axis=0)
    if Kp != K:
        Wpre = jnp.pad(Wpre, ((0, 0), (0, Kp - K)))
        bpre = jnp.pad(bpre, (0, Kp - K))
    Wd, Ws, We2 = Wpre[:fin], Wpre[fin:2 * fin], Wpre[2 * fin:]

    A, B, M3 = _proj(x, Wd, Ws, We2, c["edge"]["b"], bpre, c["edge"]["W"])

    srcs, dsts, a0s, a1s, a2s = sorted_fields
    accum = _make_sc_accum(Kp)
    S, Q, MN, MX, cnt16 = accum(srcs, dsts, a0s, a1s, a2s, hist, B, M3)
    if count is None:
        count = cnt16

    Wpost = jnp.stack([c["post"][t]["W"] for t in range(TOWERS)])
    bpost = jnp.stack([c["post"][t]["b"] for t in range(TOWERS)])
    out = _post(fin, Kp, x, A, S, Q, MN, MX, count, Wpost, bpost,
                c["lin"]["W"], c["lin"]["b"], c["ln_g"], c["ln_b"])
    return out, count


def kernel(x, edge_attr, params, edge_index, batch):
    src = edge_index[0].astype(jnp.int32)
    dst = edge_index[1].astype(jnp.int32)
    pad = E_IN_PAD - E
    src_p = jnp.pad(src, (0, pad))
    dst_p = jnp.pad(dst, (0, pad))
    attr_p = jnp.pad(edge_attr.astype(jnp.float32), ((0, pad), (0, 0))).T

    hist = _sc_hist_k()(dst_p)
    sorted_fields = _sc_permute_k()(src_p, dst_p, attr_p[0], attr_p[1],
                                    attr_p[2], hist)

    count = None
    for c in params["convs"]:
        x, count = _conv_layer(c, x, sorted_fields, hist, count)

    g = _pool(x, batch.astype(jnp.int32))
    return _head(g, params["mlp"], params["out"])


# speedup vs baseline: 5.0922x; 1.0002x over previous
import functools

import jax
import jax.numpy as jnp
import numpy as np
from jax import lax
from jax.experimental import pallas as pl
from jax.experimental.pallas import tpu as pltpu
from jax.experimental.pallas import tpu_sc as plsc

N = 10000
E = 160000
NG = 128
HIDDEN = 64
TOWERS = 4
F_OUT_T = HIDDEN // TOWERS
AVG_LOG = float(np.log(17.0))
BLK = 2000
GRID = N // BLK

NTILES = 32
BSHIFT = 5
BW = 1 << BSHIFT
NB = 320
N_PAD = NB * BW
NBPT = NB // NTILES
CH = 5024
E_IN_PAD = NTILES * CH
C3 = 256
E_PAD = E + 392
DUMP = E + 320

def _wid():
    return lax.axis_index("s") * 2 + lax.axis_index("c")


@functools.cache
def _sc_hist_k():
    return functools.partial(
        pl.kernel,
        out_type=jax.ShapeDtypeStruct((NTILES, NB), jnp.int32),
        mesh=plsc.VectorSubcoreMesh(core_axis_name="c", subcore_axis_name="s"),
        scratch_types=[
            pltpu.VMEM((CH + 16,), jnp.int32),
            pltpu.VMEM((NB,), jnp.int32),
            pltpu.SMEM((NB,), jnp.int32),
        ],
    )(_sc_hist_body)


def _sc_hist_body(dst_hbm, hist_out, dstv, histv, histm):
    w = _wid()
    base = w * CH
    cnt = jnp.minimum(CH, E - base)
    pltpu.sync_copy(dst_hbm.at[pl.ds(base, CH)], dstv.at[pl.ds(0, CH)])

    def z(b, c):
        histm[b] = 0
        return c

    lax.fori_loop(0, NB, z, 0)

    def body(i, carry):
        b = lax.shift_right_logical(dstv[pl.ds(i, 16)][0], BSHIFT)
        histm[b] = histm[b] + 1
        return carry

    lax.fori_loop(0, cnt, body, 0)

    lane = lax.iota(jnp.int32, 16)
    zero16 = jnp.zeros((16,), jnp.int32)

    def red(bg, c):
        tot = zero16
        for l in range(16):
            tot = jnp.where(lane == l, histm[bg * 16 + l], tot)
        histv[pl.ds(bg * 16, 16)] = tot
        return c

    lax.fori_loop(0, NB // 16, red, 0)
    pltpu.sync_copy(histv, hist_out.at[w])


@functools.cache
def _sc_permute_k():
    return functools.partial(
        pl.kernel,
        out_type=[
            jax.ShapeDtypeStruct((E_PAD,), jnp.int32),
            jax.ShapeDtypeStruct((E_PAD,), jnp.int32),
            jax.ShapeDtypeStruct((E_PAD,), jnp.float32),
            jax.ShapeDtypeStruct((E_PAD,), jnp.float32),
            jax.ShapeDtypeStruct((E_PAD,), jnp.float32),
        ],
        mesh=plsc.VectorSubcoreMesh(core_axis_name="c", subcore_axis_name="s"),
        scratch_types=[
            pltpu.VMEM((NTILES, NB), jnp.int32),
            pltpu.SMEM((NB,), jnp.int32),
            pltpu.VMEM((5040,), jnp.int32),
            pltpu.VMEM((5040,), jnp.int32),
            pltpu.VMEM((5040,), jnp.float32),
            pltpu.VMEM((5040,), jnp.float32),
            pltpu.VMEM((5040,), jnp.float32),
            pltpu.VMEM((45, 112), jnp.int32),
            pltpu.VMEM((256,), jnp.float32),
            pltpu.VMEM((256,), jnp.int32),
            pltpu.SemaphoreType.DMA,
        ],
    )(_sc_permute_body)


def _sc_permute_body(src_hbm, dst_hbm, a0_hbm, a1_hbm, a2_hbm, hist_hbm,
                srcs_out, dsts_out, a0s_out, a1s_out, a2s_out,
                histv, offsm, sv, dv, a0v, a1v, a2v, posv, zbuf, zbi, sem):
    w = _wid()
    base = w * CH
    cnt = jnp.minimum(CH, E - base)
    pltpu.sync_copy(hist_hbm, histv)
    pltpu.sync_copy(src_hbm.at[pl.ds(base, CH)], sv.at[pl.ds(0, CH)])
    pltpu.sync_copy(dst_hbm.at[pl.ds(base, CH)], dv.at[pl.ds(0, CH)])
    pltpu.sync_copy(a0_hbm.at[pl.ds(base, CH)], a0v.at[pl.ds(0, CH)])
    pltpu.sync_copy(a1_hbm.at[pl.ds(base, CH)], a1v.at[pl.ds(0, CH)])
    pltpu.sync_copy(a2_hbm.at[pl.ds(base, CH)], a2v.at[pl.ds(0, CH)])

    lane = lax.iota(jnp.int32, 16)
    zero16 = jnp.zeros((16,), jnp.int32)

    def ob(bg, run):
        tot16 = zero16
        mine16 = zero16

        def iw(w2, c):
            t16, m16 = c
            h16 = histv[w2, pl.ds(bg * 16, 16)]
            return (t16 + h16, m16 + jnp.where(w2 < w, h16, zero16))

        tot16, mine16 = lax.fori_loop(0, NTILES, iw, (tot16, mine16))
        for l in range(16):
            offsm[bg * 16 + l] = run + mine16[l]
            run = run + tot16[l]
        return run

    lax.fori_loop(0, NB // 16, ob, 0)

    dumpv = jnp.full((16,), DUMP, jnp.int32)

    def dump(g, carry):
        cc = g // 7
        gi = g - cc * 7
        posv[cc, pl.ds(gi * 16, 16)] = dumpv
        return carry

    lax.fori_loop(0, 315, dump, 0)

    def grp(g, carry):
        cc = g // 7
        gi = g - cc * 7
        b16 = lax.shift_right_logical(dv[pl.ds(g * 16, 16)], BSHIFT)
        pos16 = zero16
        for l in range(16):
            b = b16[l]
            p = offsm[b]
            offsm[b] = p + 1
            pos16 = jnp.where(lane == l, p, pos16)
        posv[cc, pl.ds(gi * 16, 16)] = pos16
        return carry

    lax.fori_loop(0, lax.shift_right_arithmetic(cnt, 4), grp, 0)

    pending = []
    for c in range(45):
        sl = pl.ds(c * 112, 112)
        idx = posv.at[c]
        pending.append([
            pltpu.async_copy(sv.at[sl], srcs_out.at[idx], sem),
            pltpu.async_copy(dv.at[sl], dsts_out.at[idx], sem),
            pltpu.async_copy(a0v.at[sl], a0s_out.at[idx], sem),
            pltpu.async_copy(a1v.at[sl], a1s_out.at[idx], sem),
            pltpu.async_copy(a2v.at[sl], a2s_out.at[idx], sem),
        ])
        if c >= 8:
            for d in pending[c - 8]:
                d.wait()
    for grp in pending[-8:]:
        for d in grp:
            d.wait()

    @pl.when(w == 0)
    def _():
        for i in range(16):
            zbuf[pl.ds(i * 16, 16)] = jnp.zeros((16,), jnp.float32)
            zbi[pl.ds(i * 16, 16)] = jnp.zeros((16,), jnp.int32)
        for o in (E, E + 64):
            pltpu.sync_copy(zbi, srcs_out.at[pl.ds(o, 256)])
            pltpu.sync_copy(zbi, dsts_out.at[pl.ds(o, 256)])
            pltpu.sync_copy(zbuf, a0s_out.at[pl.ds(o, 256)])
            pltpu.sync_copy(zbuf, a1s_out.at[pl.ds(o, 256)])
            pltpu.sync_copy(zbuf, a2s_out.at[pl.ds(o, 256)])


@functools.cache
def _make_sc_accum(K):
    NJ = K // 16

    @functools.partial(
        pl.kernel,
        out_type=[
            jax.ShapeDtypeStruct((N_PAD, K), jnp.float32),
            jax.ShapeDtypeStruct((N_PAD, K), jnp.float32),
            jax.ShapeDtypeStruct((N_PAD, K), jnp.float32),
            jax.ShapeDtypeStruct((N_PAD, K), jnp.float32),
            jax.ShapeDtypeStruct((N_PAD, 16), jnp.float32),
        ],
        mesh=plsc.VectorSubcoreMesh(core_axis_name="c", subcore_axis_name="s"),
        scratch_types=[
            pltpu.VMEM((NTILES, NB), jnp.int32),
            pltpu.SMEM((NB + 1,), jnp.int32),
            pltpu.VMEM((288,), jnp.int32),
            pltpu.VMEM((288,), jnp.int32),
            pltpu.VMEM((288,), jnp.float32),
            pltpu.VMEM((288,), jnp.float32),
            pltpu.VMEM((288,), jnp.float32),
            pltpu.VMEM((272, K), jnp.float32),
            pltpu.VMEM((3, K), jnp.float32),
            pltpu.VMEM((BW, K), jnp.float32),
            pltpu.VMEM((BW, K), jnp.float32),
            pltpu.VMEM((BW, K), jnp.float32),
            pltpu.VMEM((BW, K), jnp.float32),
            pltpu.VMEM((BW, 16), jnp.float32),
            pltpu.SemaphoreType.DMA,
        ],
    )
    def _sc_accum(srcs_hbm, dsts_hbm, a0_hbm, a1_hbm, a2_hbm, hist_hbm,
                  B_hbm, m3_hbm, S_out, Q_out, MN_out, MX_out, cnt_out,
                  histv, prefm, sv, dv, a0v, a1v, a2v, rowsv, m3v,
                  accS, accQ, accMN, accMX, accC, sem):
        w = _wid()
        pltpu.sync_copy(hist_hbm, histv)
        pltpu.sync_copy(m3_hbm, m3v)

        zero16i = jnp.zeros((16,), jnp.int32)

        def pb(bg, run):
            def iw(w2, t16):
                return t16 + histv[w2, pl.ds(bg * 16, 16)]

            tot16 = lax.fori_loop(0, NTILES, iw, zero16i)
            for l in range(16):
                prefm[bg * 16 + l] = run
                run = run + tot16[l]
            return run

        run = lax.fori_loop(0, NB // 16, pb, 0)
        prefm[NB] = run

        zero16 = jnp.zeros((16,), jnp.float32)
        big16 = jnp.full((16,), 1e30, jnp.float32)
        ones16 = jnp.ones((16,), jnp.float32)
        m3c = [[m3v[r, pl.ds(j * 16, 16)] for r in range(3)]
               for j in range(NJ)]

        def tbody(t, tcarry):
            b = w + NTILES * t
            start = prefm[b]
            end = prefm[b + 1]

            def zr(r, carry):
                for j in range(NJ):
                    accS[r, pl.ds(j * 16, 16)] = zero16
                    accQ[r, pl.ds(j * 16, 16)] = zero16
                    accMN[r, pl.ds(j * 16, 16)] = big16
                    accMX[r, pl.ds(j * 16, 16)] = -big16
                accC[r, pl.ds(0, 16)] = zero16
                return carry

            lax.fori_loop(0, BW, zr, 0)

            def chunk_body(k, kcarry):
                cbase = start + k * C3
                a8 = pl.multiple_of(lax.bitwise_and(cbase, ~15), 16)
                off = cbase - a8
                valid = jnp.minimum(C3, end - cbase)
                pltpu.sync_copy(srcs_hbm.at[pl.ds(a8, 288)], sv)
                g1 = pltpu.async_copy(B_hbm.at[sv.at[pl.ds(0, 128)]],
                                      rowsv.at[pl.ds(0, 128)], sem)
                g2 = pltpu.async_copy(B_hbm.at[sv.at[pl.ds(128, 128)]],
                                      rowsv.at[pl.ds(128, 128)], sem)
                g3 = pltpu.async_copy(B_hbm.at[sv.at[pl.ds(256, 16)]],
                                      rowsv.at[pl.ds(256, 16)], sem)
                f1 = pltpu.async_copy(dsts_hbm.at[pl.ds(a8, 288)], dv, sem)
                f2 = pltpu.async_copy(a0_hbm.at[pl.ds(a8, 288)], a0v, sem)
                f3 = pltpu.async_copy(a1_hbm.at[pl.ds(a8, 288)], a1v, sem)
                f4 = pltpu.async_copy(a2_hbm.at[pl.ds(a8, 288)], a2v, sem)
                g1.wait()
                g2.wait()
                g3.wait()
                f1.wait()
                f2.wait()
                f3.wait()
                f4.wait()

                def edge(i, carry):
                    ii = off + i
                    dl = lax.bitwise_and(dv[pl.ds(ii, 16)][0], BW - 1)
                    a0 = a0v[pl.ds(ii, 16)][0]
                    a1 = a1v[pl.ds(ii, 16)][0]
                    a2 = a2v[pl.ds(ii, 16)][0]
                    for j in range(NJ):
                        sl = pl.ds(j * 16, 16)
                        m = (rowsv[ii, sl] + a0 * m3c[j][0]
                             + a1 * m3c[j][1] + a2 * m3c[j][2])
                        plsc.addupdate(accS.at[dl, sl], m)
                        plsc.addupdate(accQ.at[dl, sl], m * m)
                        accMN[dl, sl] = jnp.minimum(accMN[dl, sl], m)
                        accMX[dl, sl] = jnp.maximum(accMX[dl, sl], m)
                    plsc.addupdate(accC.at[dl, pl.ds(0, 16)], ones16)
                    return carry

                lax.fori_loop(0, valid, edge, 0)
                return kcarry

            nch = lax.shift_right_arithmetic(end - start + C3 - 1, 8)
            lax.fori_loop(0, nch, chunk_body, 0)

            rsl = pl.ds(b * BW, BW)
            pltpu.sync_copy(accS, S_out.at[rsl])
            pltpu.sync_copy(accQ, Q_out.at[rsl])
            pltpu.sync_copy(accMN, MN_out.at[rsl])
            pltpu.sync_copy(accMX, MX_out.at[rsl])
            pltpu.sync_copy(accC, cnt_out.at[rsl])
            return tcarry

        lax.fori_loop(0, NBPT, tbody, 0)

    return _sc_accum


def _layer_norm(x, g, b):
    mu = jnp.mean(x, axis=-1, keepdims=True)
    var = jnp.mean((x - mu) ** 2, axis=-1, keepdims=True)
    return (x - mu) / jnp.sqrt(var + 1e-5) * g + b


def _proj_kernel(xb, Wd, Ws, We2, be, bpre, We, A_out, B_out, M3_out):
    c0 = be[...] @ We2[...]
    A_out[...] = xb[...] @ Wd[...] + bpre[...] + c0
    B_out[...] = xb[...] @ Ws[...]
    M3_out[...] = We[...] @ We2[...]


def _proj(x, Wd, Ws, We2, be, bpre, We):
    fin = x.shape[1]
    Kp = Wd.shape[1]
    return pl.pallas_call(
        _proj_kernel,
        grid=(GRID,),
        in_specs=[
            pl.BlockSpec((BLK, fin), lambda i: (i, 0)),
            pl.BlockSpec((fin, Kp), lambda i: (0, 0)),
            pl.BlockSpec((fin, Kp), lambda i: (0, 0)),
            pl.BlockSpec((fin, Kp), lambda i: (0, 0)),
            pl.BlockSpec((fin,), lambda i: (0,)),
            pl.BlockSpec((Kp,), lambda i: (0,)),
            pl.BlockSpec((3, fin), lambda i: (0, 0)),
        ],
        out_specs=[
            pl.BlockSpec((BLK, Kp), lambda i: (i, 0)),
            pl.BlockSpec((BLK, Kp), lambda i: (i, 0)),
            pl.BlockSpec((3, Kp), lambda i: (0, 0)),
        ],
        out_shape=[
            jax.ShapeDtypeStruct((N, Kp), jnp.float32),
            jax.ShapeDtypeStruct((N, Kp), jnp.float32),
            jax.ShapeDtypeStruct((3, Kp), jnp.float32),
        ],
    )(x, Wd, Ws, We2, be, bpre, We)


def _post_kernel(fin, xb, Ab, Sb, Qb, MNb, MXb, count_ref, Wpost, bpost,
                 Wlin, blin, ln_g, ln_b, out_ref):
    K = 4 * fin
    cntf = count_ref[...][:, 0:1]
    has = cntf > 0.0
    cnt = jnp.maximum(cntf, 1.0)
    logdeg = jnp.log(cnt + 1.0)
    amp = logdeg / AVG_LOG
    att = AVG_LOG / logdeg

    A = Ab[...]
    S = Sb[...]
    Q = Qb[...]
    mean = jnp.where(has, A + S / cnt, 0.0)
    var = jnp.maximum(Q / cnt - (S / cnt) ** 2, 0.0)
    std = jnp.sqrt(jnp.where(has, var, 0.0) + 1e-5)
    mn = jnp.where(has, A + MNb[...], 0.0)
    mx = jnp.where(has, A + MXb[...], 0.0)

    y = jnp.zeros((BLK, HIDDEN), jnp.float32)
    bias = blin[...]
    stats = (mean, mn, mx, std)
    for t in range(TOWERS):
        Wl_t = Wlin[t * F_OUT_T:(t + 1) * F_OUT_T, :]
        Wp = Wpost[t]
        Px_t = Wp[0:fin, :] @ Wl_t
        y += xb[...] @ Px_t
        bias += bpost[t] @ Wl_t
        for g in range(3):
            scale = (1.0, amp, att)[g]
            acc = jnp.zeros((BLK, HIDDEN), jnp.float32)
            for s in range(4):
                lo = fin + g * K + s * fin
                Wrows = Wp[lo:lo + fin, :] @ Wl_t
                acc += stats[s][:, t * fin:(t + 1) * fin] @ Wrows
            y = y + acc * scale
    y = y + bias
    out_ref[...] = jax.nn.relu(_layer_norm(y, ln_g[...], ln_b[...]))


def _post(fin, Kp, x, A, S, Q, MN, MX, count, Wpost, bpost, Wlin, blin,
          ln_g, ln_b):
    return pl.pallas_call(
        functools.partial(_post_kernel, fin),
        grid=(GRID,),
        in_specs=[
            pl.BlockSpec((BLK, fin), lambda i: (i, 0)),
            pl.BlockSpec((BLK, Kp), lambda i: (i, 0)),
            pl.BlockSpec((BLK, Kp), lambda i: (i, 0)),
            pl.BlockSpec((BLK, Kp), lambda i: (i, 0)),
            pl.BlockSpec((BLK, Kp), lambda i: (i, 0)),
            pl.BlockSpec((BLK, Kp), lambda i: (i, 0)),
            pl.BlockSpec((BLK, 16), lambda i: (i, 0)),
            pl.BlockSpec((TOWERS, 13 * fin, F_OUT_T), lambda i: (0, 0, 0)),
            pl.BlockSpec((TOWERS, F_OUT_T), lambda i: (0, 0)),
            pl.BlockSpec((HIDDEN, HIDDEN), lambda i: (0, 0)),
            pl.BlockSpec((HIDDEN,), lambda i: (0,)),
            pl.BlockSpec((HIDDEN,), lambda i: (0,)),
            pl.BlockSpec((HIDDEN,), lambda i: (0,)),
        ],
        out_specs=pl.BlockSpec((BLK, HIDDEN), lambda i: (i, 0)),
        out_shape=jax.ShapeDtypeStruct((N, HIDDEN), jnp.float32),
    )(x, A, S, Q, MN, MX, count, Wpost, bpost, Wlin, blin, ln_g, ln_b)


def _pool_kernel(xb, batch_ref, g_out):
    i = pl.program_id(0)

    @pl.when(i == 0)
    def _():
        g_out[...] = jnp.zeros_like(g_out)

    b = batch_ref[...].reshape(1, BLK)
    onehot = (jax.lax.broadcasted_iota(jnp.int32, (NG, BLK), 0)
              == b).astype(jnp.float32)
    g_out[...] += onehot @ xb[...]


def _pool(x, batch):
    return pl.pallas_call(
        _pool_kernel,
        grid=(GRID,),
        in_specs=[
            pl.BlockSpec((BLK, HIDDEN), lambda i: (i, 0)),
            pl.BlockSpec((1, 1, BLK), lambda i: (i, 0, 0)),
        ],
        out_specs=pl.BlockSpec((NG, HIDDEN), lambda i: (0, 0)),
        out_shape=jax.ShapeDtypeStruct((NG, HIDDEN), jnp.float32),
    )(x, batch.reshape(GRID, 1, BLK))


def _head_kernel(g_ref, l1w, l1b, ln1g, ln1b, l2w, l2b, ln2g, ln2b,
                 o1w, o1b, o2w, o2b, o3w, o3b, out_ref):
    g = g_ref[...]
    g = jax.nn.relu(_layer_norm(g @ l1w[...] + l1b[...], ln1g[...], ln1b[...]))
    g = jax.nn.relu(_layer_norm(g @ l2w[...] + l2b[...], ln2g[...], ln2b[...]))
    g = jax.nn.relu(g @ o1w[...] + o1b[...])
    g = jax.nn.relu(g @ o2w[...] + o2b[...])
    g = g @ o3w[...] + o3b[...]
    out_ref[...] = jnp.abs(g)


def _head(g, m, o):
    args = (g, m["l1"]["W"], m["l1"]["b"], m["ln1_g"], m["ln1_b"],
            m["l2"]["W"], m["l2"]["b"], m["ln2_g"], m["ln2_b"],
            o["o1"]["W"], o["o1"]["b"], o["o2"]["W"], o["o2"]["b"],
            o["o3"]["W"], o["o3"]["b"])
    return pl.pallas_call(
        _head_kernel,
        out_shape=jax.ShapeDtypeStruct((NG, 3), jnp.float32),
    )(*args)


def _conv_layer(c, x, sorted_fields, hist, count):
    fin = x.shape[1]
    K = TOWERS * fin
    Kp = 128 if K == 36 else K

    Wpre = jnp.concatenate([c["pre"][t]["W"] for t in range(TOWERS)], axis=1)
    bpre = jnp.concatenate([c["pre"][t]["b"] for t in range(TOWERS)], axis=0)
    if Kp != K:
        Wpre = jnp.pad(Wpre, ((0, 0), (0, Kp - K)))
        bpre = jnp.pad(bpre, (0, Kp - K))
    Wd, Ws, We2 = Wpre[:fin], Wpre[fin:2 * fin], Wpre[2 * fin:]

    A, B, M3 = _proj(x, Wd, Ws, We2, c["edge"]["b"], bpre, c["edge"]["W"])

    srcs, dsts, a0s, a1s, a2s = sorted_fields
    accum = _make_sc_accum(Kp)
    S, Q, MN, MX, cnt16 = accum(srcs, dsts, a0s, a1s, a2s, hist, B, M3)
    if count is None:
        count = cnt16

    Wpost = jnp.stack([c["post"][t]["W"] for t in range(TOWERS)])
    bpost = jnp.stack([c["post"][t]["b"] for t in range(TOWERS)])
    out = _post(fin, Kp, x, A, S, Q, MN, MX, count, Wpost, bpost,
                c["lin"]["W"], c["lin"]["b"], c["ln_g"], c["ln_b"])
    return out, count


def kernel(x, edge_attr, params, edge_index, batch):
    src = edge_index[0].astype(jnp.int32)
    dst = edge_index[1].astype(jnp.int32)
    pad = E_IN_PAD - E
    src_p = jnp.pad(src, (0, pad))
    dst_p = jnp.pad(dst, (0, pad))
    attr_p = jnp.pad(edge_attr.astype(jnp.float32), ((0, pad), (0, 0))).T

    hist = _sc_hist_k()(dst_p)
    sorted_fields = _sc_permute_k()(src_p, dst_p, attr_p[0], attr_p[1],
                                    attr_p[2], hist)

    count = None
    for c in params["convs"]:
        x, count = _conv_layer(c, x, sorted_fields, hist, count)

    g = _pool(x, batch.astype(jnp.int32))
    return _head(g, params["mlp"], params["out"])
